# Initial kernel scaffold; baseline (speedup 1.0000x reference)
import functools
import math

import jax
import jax.numpy as jnp
from jax import lax
from jax.experimental import pallas as pl
from jax.experimental.pallas import tpu as pltpu
from jax.experimental.pallas import tpu_sc as plsc

H = 128
EC = 16
TW = 144
NW = 32
KCH = 128

_F32 = jnp.float32


def _silu(v):
    return v * (1.0 / (1.0 + jnp.exp(-v)))


def _dot(a, b):
    return jnp.dot(a, b, preferred_element_type=_F32)


def _sc_gather(table, col, row):
    e = col.shape[0]
    per_w = e // NW
    nfull = per_w // KCH
    rem = per_w - nfull * KCH
    mesh = plsc.VectorSubcoreMesh(core_axis_name="c", subcore_axis_name="s")

    scratch = [
        pltpu.VMEM((KCH,), jnp.int32),
        pltpu.VMEM((KCH, TW), _F32),
        pltpu.SemaphoreType.DMA,
    ]
    if rem:
        scratch += [pltpu.VMEM((rem,), jnp.int32), pltpu.VMEM((rem, TW), _F32)]

    @functools.partial(
        pl.kernel,
        mesh=mesh,
        out_type=(
            jax.ShapeDtypeStruct((e, TW), _F32),
            jax.ShapeDtypeStruct((e, TW), _F32),
        ),
        scratch_types=scratch,
    )
    def k(table_hbm, col_hbm, row_hbm, xi_hbm, xj_hbm, idx_v, rows_v, sem,
          *rest):
        wid = lax.axis_index("s") * 2 + lax.axis_index("c")
        base = wid * per_w
        for idx_hbm, out_hbm in ((col_hbm, xi_hbm), (row_hbm, xj_hbm)):
            def body(ci, _, idx_hbm=idx_hbm, out_hbm=out_hbm):
                off = base + ci * KCH
                pltpu.sync_copy(idx_hbm.at[pl.ds(off, KCH)], idx_v)
                pltpu.async_copy(table_hbm.at[idx_v], rows_v, sem).wait()
                pltpu.sync_copy(rows_v, out_hbm.at[pl.ds(off, KCH)])
                return 0
            lax.fori_loop(0, nfull, body, 0)
            if rem:
                idx_r, rows_r = rest
                off = base + nfull * KCH
                pltpu.sync_copy(idx_hbm.at[pl.ds(off, rem)], idx_r)
                pltpu.async_copy(table_hbm.at[idx_r], rows_r, sem).wait()
                pltpu.sync_copy(rows_r, out_hbm.at[pl.ds(off, rem)])

    return k(table, col, row)


def _sc_scatter(m, col, n, zeros_tab):
    e = col.shape[0]
    per_w = e // NW
    nfull = per_w // KCH
    rem = per_w - nfull * KCH
    rows_per_tile = n // 16
    mesh = plsc.VectorSubcoreMesh(core_axis_name="c", subcore_axis_name="s")

    scratch = [
        pltpu.VMEM((KCH,), jnp.int32),
        pltpu.VMEM((KCH, TW), _F32),
        pltpu.VMEM_SHARED((n, TW), _F32),
        pltpu.SemaphoreType.DMA,
    ]
    if rem:
        scratch += [pltpu.VMEM((rem,), jnp.int32), pltpu.VMEM((rem, TW), _F32)]

    @functools.partial(
        pl.kernel,
        mesh=mesh,
        out_type=jax.ShapeDtypeStruct((2, n, TW), _F32),
        scratch_types=scratch,
    )
    def k(m_hbm, col_hbm, zeros_hbm, parts_hbm, idx_v, rows_v, acc, sem,
          *rest):
        c = lax.axis_index("c")
        s = lax.axis_index("s")
        wid = s * 2 + c
        base = wid * per_w
        zoff = s * rows_per_tile
        pltpu.sync_copy(zeros_hbm.at[pl.ds(zoff, rows_per_tile)],
                        acc.at[pl.ds(zoff, rows_per_tile)])
        plsc.subcore_barrier()

        def body(ci, _):
            off = base + ci * KCH
            pltpu.sync_copy(col_hbm.at[pl.ds(off, KCH)], idx_v)
            pltpu.sync_copy(m_hbm.at[pl.ds(off, KCH)], rows_v)
            pltpu.sync_copy(rows_v, acc.at[idx_v], add=True)
            return 0
        lax.fori_loop(0, nfull, body, 0)
        if rem:
            idx_r, rows_r = rest
            off = base + nfull * KCH
            pltpu.sync_copy(col_hbm.at[pl.ds(off, rem)], idx_r)
            pltpu.sync_copy(m_hbm.at[pl.ds(off, rem)], rows_r)
            pltpu.sync_copy(rows_r, acc.at[idx_r], add=True)
        plsc.subcore_barrier()
        pltpu.sync_copy(acc.at[pl.ds(zoff, rows_per_tile)],
                        parts_hbm.at[c, pl.ds(zoff, rows_per_tile)])

    return k(m, col, zeros_tab)


def _full(shape):
    return pl.BlockSpec(shape, lambda i: (0,) * len(shape))


def _tc_prologue(at2, b2, pos, t2, p):
    n = at2.shape[0]
    bn = 2000
    nat = p["atom_emb"].shape[0]
    nb = t2.shape[0]

    def body(at_ref, b_ref, pos_ref, t_ref, emb_w, t1w, t1b, t2w, t2b,
             out_ref):
        tcol = t_ref[...]
        io = lax.broadcasted_iota(_F32, (1, H // 2), 1)
        freqs = jnp.exp(io * (-math.log(10000.0) / (H // 2 - 1)))
        args = tcol * freqs
        emb = jnp.concatenate([jnp.sin(args), jnp.cos(args)], axis=1)
        temb = _dot(_silu(_dot(emb, t1w[...]) + t1b[...]), t2w[...]) + t2b[...]
        oh_a = (at_ref[...] == lax.broadcasted_iota(jnp.int32, (bn, nat), 1)
                ).astype(_F32)
        oh_b = (b_ref[...] == lax.broadcasted_iota(jnp.int32, (bn, nb), 1)
                ).astype(_F32)
        h0 = _dot(oh_a, emb_w[...]) + _dot(oh_b, temb)
        out_ref[...] = jnp.concatenate(
            [h0, pos_ref[...], jnp.zeros((bn, TW - H - 3), _F32)], axis=1)

    return pl.pallas_call(
        body,
        grid=(n // bn,),
        in_specs=[
            pl.BlockSpec((bn, 1), lambda i: (i, 0)),
            pl.BlockSpec((bn, 1), lambda i: (i, 0)),
            pl.BlockSpec((bn, 3), lambda i: (i, 0)),
            _full((nb, 1)),
            _full(p["atom_emb"].shape),
            _full((H, H)), _full((1, H)),
            _full((H, H)), _full((1, H)),
        ],
        out_specs=pl.BlockSpec((bn, TW), lambda i: (i, 0)),
        out_shape=jax.ShapeDtypeStruct((n, TW), _F32),
    )(at2, b2, pos, t2, p["atom_emb"],
      p["time1"]["W"], p["time1"]["b"].reshape(1, H),
      p["time2"]["W"], p["time2"]["b"].reshape(1, H))


def _edge_tail(xi, xj, ea, rel, dist, lw, m_ref):
    be = xi.shape[0]
    cat1 = jnp.concatenate([xi, xj, ea], axis=1)
    e1 = _silu(_dot(cat1, lw["e1w"][...]) + lw["e1b"][...])
    ef = _dot(e1, lw["e2w"][...]) + lw["e2b"][...]
    cat2 = jnp.concatenate([xi, xj, ef], axis=1)
    n1 = _silu(_dot(cat2, lw["n1w"][...]) + lw["n1b"][...])
    nm = _dot(n1, lw["n2w"][...]) + lw["n2b"][...]
    cat3 = jnp.concatenate([nm, ef], axis=1)
    c1 = _silu(_dot(cat3, lw["c1w"][...]) + lw["c1b"][...])
    cw = _dot(c1, lw["c2w"][...])
    cm = cw * (rel / (dist + 1e-8))
    m_ref[...] = jnp.concatenate(
        [nm, cm, jnp.zeros((be, TW - H - 3), _F32)], axis=1)


def _layer_weights(lp):
    return {
        "e1w": lp["edge1"]["W"], "e1b": lp["edge1"]["b"].reshape(1, EC),
        "e2w": lp["edge2"]["W"], "e2b": lp["edge2"]["b"].reshape(1, EC),
        "n1w": lp["node1"]["W"], "n1b": lp["node1"]["b"].reshape(1, 2 * H),
        "n2w": lp["node2"]["W"], "n2b": lp["node2"]["b"].reshape(1, H),
        "c1w": lp["coord1"]["W"], "c1b": lp["coord1"]["b"].reshape(1, H),
        "c2w": lp["coord2"]["W"],
    }


_LW_ORDER = ("e1w", "e1b", "e2w", "e2b", "n1w", "n1b", "n2w", "n2b",
             "c1w", "c1b", "c2w")


def _tc_edge1(Xi, Xj, lp, p):
    e = Xi.shape[0]
    be = 1000
    lw = _layer_weights(lp)

    def body(xi_ref, xj_ref, *refs):
        lwr = dict(zip(_LW_ORDER, refs[:11]))
        ec1w, ec1b, ec2w, ec2b, m_ref, ea_ref = refs[11:]
        Xib = xi_ref[...]
        Xjb = xj_ref[...]
        xi = Xib[:, :H]
        xj = Xjb[:, :H]
        rel = Xjb[:, H:H + 3] - Xib[:, H:H + 3]
        dist = jnp.sqrt(jnp.sum(rel * rel, axis=1, keepdims=True))
        ea0 = _silu(dist * ec1w[...] + ec1b[...])
        ea = _dot(ea0, ec2w[...]) + ec2b[...]
        ea_ref[...] = ea
        _edge_tail(xi, xj, ea, rel, dist, lwr, m_ref)

    lw_vals = [lw[k] for k in _LW_ORDER]
    return pl.pallas_call(
        body,
        grid=(e // be,),
        in_specs=[pl.BlockSpec((be, TW), lambda i: (i, 0))] * 2
        + [_full(v.shape) for v in lw_vals]
        + [_full((1, EC))] * 2 + [_full((EC, EC)), _full((1, EC))],
        out_specs=[pl.BlockSpec((be, TW), lambda i: (i, 0)),
                   pl.BlockSpec((be, EC), lambda i: (i, 0))],
        out_shape=[jax.ShapeDtypeStruct((e, TW), _F32),
                   jax.ShapeDtypeStruct((e, EC), _F32)],
    )(Xi, Xj, *lw_vals,
      p["edge_enc1"]["W"].reshape(1, EC), p["edge_enc1"]["b"].reshape(1, EC),
      p["edge_enc2"]["W"], p["edge_enc2"]["b"].reshape(1, EC))


def _tc_edge2(Xi, Xj, ea_arr, lp):
    e = Xi.shape[0]
    be = 1000
    lw = _layer_weights(lp)

    def body(xi_ref, xj_ref, ea_ref, *refs):
        lwr = dict(zip(_LW_ORDER, refs[:11]))
        m_ref = refs[11]
        Xib = xi_ref[...]
        Xjb = xj_ref[...]
        xi = Xib[:, :H]
        xj = Xjb[:, :H]
        rel = Xjb[:, H:H + 3] - Xib[:, H:H + 3]
        dist = jnp.sqrt(jnp.sum(rel * rel, axis=1, keepdims=True))
        _edge_tail(xi, xj, ea_ref[...], rel, dist, lwr, m_ref)

    lw_vals = [lw[k] for k in _LW_ORDER]
    return pl.pallas_call(
        body,
        grid=(e // be,),
        in_specs=[pl.BlockSpec((be, TW), lambda i: (i, 0))] * 2
        + [pl.BlockSpec((be, EC), lambda i: (i, 0))]
        + [_full(v.shape) for v in lw_vals],
        out_specs=pl.BlockSpec((be, TW), lambda i: (i, 0)),
        out_shape=jax.ShapeDtypeStruct((e, TW), _F32),
    )(Xi, Xj, ea_arr, *lw_vals)


def _tc_update(table, p0, p1):
    n = table.shape[0]
    bn = 2000

    def body(t_ref, a_ref, b_ref, o_ref):
        o_ref[...] = t_ref[...] + a_ref[...] + b_ref[...]

    return pl.pallas_call(
        body,
        grid=(n // bn,),
        in_specs=[pl.BlockSpec((bn, TW), lambda i: (i, 0))] * 3,
        out_specs=pl.BlockSpec((bn, TW), lambda i: (i, 0)),
        out_shape=jax.ShapeDtypeStruct((n, TW), _F32),
    )(table, p0, p1)


def _tc_epilogue(table, p0, p1, p):
    n = table.shape[0]
    bn = 2000
    nat_out = p["atom2"]["W"].shape[1]

    def body(t_ref, a_ref, b_ref, wp1, bp1, wp2, bp2, wa1, ba1, wa2, ba2,
             pn_ref, al_ref):
        tt = t_ref[...] + a_ref[...] + b_ref[...]
        h = tt[:, :H]
        pn_ref[...] = _dot(_silu(_dot(h, wp1[...]) + bp1[...]),
                           wp2[...]) + bp2[...]
        al_ref[...] = _dot(_silu(_dot(h, wa1[...]) + ba1[...]),
                           wa2[...]) + ba2[...]

    return pl.pallas_call(
        body,
        grid=(n // bn,),
        in_specs=[pl.BlockSpec((bn, TW), lambda i: (i, 0))] * 3
        + [_full((H, H)), _full((1, H)), _full((H, 3)), _full((1, 3)),
           _full((H, H)), _full((1, H)), _full((H, nat_out)),
           _full((1, nat_out))],
        out_specs=[pl.BlockSpec((bn, 3), lambda i: (i, 0)),
                   pl.BlockSpec((bn, nat_out), lambda i: (i, 0))],
        out_shape=[jax.ShapeDtypeStruct((n, 3), _F32),
                   jax.ShapeDtypeStruct((n, nat_out), _F32)],
    )(table, p0, p1,
      p["pos1"]["W"], p["pos1"]["b"].reshape(1, H),
      p["pos2"]["W"], p["pos2"]["b"].reshape(1, 3),
      p["atom1"]["W"], p["atom1"]["b"].reshape(1, H),
      p["atom2"]["W"], p["atom2"]["b"].reshape(1, nat_out))


def kernel(x, pos, params, atom_type, batch, t, edge_index):
    del x
    p = params
    n = pos.shape[0]
    row = edge_index[0].astype(jnp.int32)
    col = edge_index[1].astype(jnp.int32)
    at2 = atom_type.astype(jnp.int32).reshape(n, 1)
    b2 = batch.astype(jnp.int32).reshape(n, 1)
    t2 = t.astype(_F32).reshape(-1, 1)
    zeros_tab = jnp.zeros((n, TW), _F32)

    table = _tc_prologue(at2, b2, pos.astype(_F32), t2, p)
    ea = None
    num_layers = len(p["layers"])
    for li, lp in enumerate(p["layers"]):
        Xi, Xj = _sc_gather(table, col, row)
        if li == 0:
            M, ea = _tc_edge1(Xi, Xj, lp, p)
        else:
            M = _tc_edge2(Xi, Xj, ea, lp)
        parts = _sc_scatter(M, col, n, zeros_tab)
        if li < num_layers - 1:
            table = _tc_update(table, parts[0], parts[1])
        else:
            out = _tc_epilogue(table, parts[0], parts[1], p)
    return out


# speedup vs baseline: 2.6372x; 2.6372x over previous
import functools
import math

import jax
import jax.numpy as jnp
from jax import lax
from jax.experimental import pallas as pl
from jax.experimental.pallas import tpu as pltpu
from jax.experimental.pallas import tpu_sc as plsc

H = 128
EC = 16
PW = 128
NW = 32
KCH = 128
NPAD = 10240

_F32 = jnp.float32


def _silu(v):
    return v * (1.0 / (1.0 + jnp.exp(-v)))


def _dot(a, b):
    return jnp.dot(a, b, preferred_element_type=_F32)


def _sc_gather(table, col, row, pos_tab=None):
    e = col.shape[0]
    per_w = e // NW
    nfull = per_w // KCH
    rem = per_w - nfull * KCH
    with_pos = pos_tab is not None
    mesh = plsc.VectorSubcoreMesh(core_axis_name="c", subcore_axis_name="s")

    scratch = [
        pltpu.VMEM((KCH,), jnp.int32),
        pltpu.VMEM((KCH, H), _F32),
        pltpu.SemaphoreType.DMA,
    ]
    out_type = [
        jax.ShapeDtypeStruct((e, H), _F32),
        jax.ShapeDtypeStruct((e, H), _F32),
    ]
    if with_pos:
        scratch.append(pltpu.VMEM((KCH, PW), _F32))
        out_type += [
            jax.ShapeDtypeStruct((e, PW), _F32),
            jax.ShapeDtypeStruct((e, PW), _F32),
        ]
    if rem:
        scratch += [pltpu.VMEM((rem,), jnp.int32), pltpu.VMEM((rem, H), _F32)]
        if with_pos:
            scratch.append(pltpu.VMEM((rem, PW), _F32))

    @functools.partial(
        pl.kernel,
        mesh=mesh,
        out_type=tuple(out_type),
        scratch_types=scratch,
    )
    def k(*refs):
        if with_pos:
            (table_hbm, pos_hbm, col_hbm, row_hbm,
             xi_hbm, xj_hbm, pi_hbm, pj_hbm) = refs[:8]
            rest = refs[8:]
            idx_v, rows_v, sem, prow_v = rest[:4]
            rest = rest[4:]
        else:
            table_hbm, col_hbm, row_hbm, xi_hbm, xj_hbm = refs[:5]
            rest = refs[5:]
            idx_v, rows_v, sem = rest[:3]
            rest = rest[3:]
        wid = lax.axis_index("s") * 2 + lax.axis_index("c")
        base = wid * per_w
        if with_pos:
            streams = ((col_hbm, xi_hbm, pi_hbm), (row_hbm, xj_hbm, pj_hbm))
        else:
            streams = ((col_hbm, xi_hbm, None), (row_hbm, xj_hbm, None))
        for idx_hbm, out_hbm, pout_hbm in streams:
            def body(ci, _, idx_hbm=idx_hbm, out_hbm=out_hbm,
                     pout_hbm=pout_hbm):
                off = base + ci * KCH
                pltpu.sync_copy(idx_hbm.at[pl.ds(off, KCH)], idx_v)
                pltpu.async_copy(table_hbm.at[idx_v], rows_v, sem).wait()
                pltpu.sync_copy(rows_v, out_hbm.at[pl.ds(off, KCH)])
                if with_pos:
                    pltpu.async_copy(pos_hbm.at[idx_v], prow_v, sem).wait()
                    pltpu.sync_copy(prow_v, pout_hbm.at[pl.ds(off, KCH)])
                return 0
            lax.fori_loop(0, nfull, body, 0)
            if rem:
                if with_pos:
                    idx_r, rows_r, prow_r = rest
                else:
                    idx_r, rows_r = rest
                off = base + nfull * KCH
                pltpu.sync_copy(idx_hbm.at[pl.ds(off, rem)], idx_r)
                pltpu.async_copy(table_hbm.at[idx_r], rows_r, sem).wait()
                pltpu.sync_copy(rows_r, out_hbm.at[pl.ds(off, rem)])
                if with_pos:
                    pltpu.async_copy(pos_hbm.at[idx_r], prow_r, sem).wait()
                    pltpu.sync_copy(prow_r, pout_hbm.at[pl.ds(off, rem)])

    if with_pos:
        return k(table, pos_tab, col, row)
    return k(table, col, row)


def _sc_scatter(m, col, zeros_tab):
    e = col.shape[0]
    per_w = e // NW
    nfull = per_w // KCH
    rem = per_w - nfull * KCH
    rows_per_tile = NPAD // 16
    mesh = plsc.VectorSubcoreMesh(core_axis_name="c", subcore_axis_name="s")

    scratch = [
        pltpu.VMEM((KCH,), jnp.int32),
        pltpu.VMEM((KCH, H), _F32),
        pltpu.VMEM_SHARED((NPAD, H), _F32),
        pltpu.SemaphoreType.DMA,
    ]
    if rem:
        scratch += [pltpu.VMEM((rem,), jnp.int32), pltpu.VMEM((rem, H), _F32)]

    @functools.partial(
        pl.kernel,
        mesh=mesh,
        out_type=jax.ShapeDtypeStruct((2, NPAD, H), _F32),
        scratch_types=scratch,
    )
    def k(m_hbm, col_hbm, zeros_hbm, parts_hbm, idx_v, rows_v, acc, sem,
          *rest):
        c = lax.axis_index("c")
        s = lax.axis_index("s")
        wid = s * 2 + c
        base = wid * per_w
        zoff = s * rows_per_tile
        pltpu.sync_copy(zeros_hbm.at[pl.ds(zoff, rows_per_tile)],
                        acc.at[pl.ds(zoff, rows_per_tile)])
        plsc.subcore_barrier()

        def body(ci, _):
            off = base + ci * KCH
            pltpu.sync_copy(col_hbm.at[pl.ds(off, KCH)], idx_v)
            pltpu.sync_copy(m_hbm.at[pl.ds(off, KCH)], rows_v)
            pltpu.sync_copy(rows_v, acc.at[idx_v], add=True)
            return 0
        lax.fori_loop(0, nfull, body, 0)
        if rem:
            idx_r, rows_r = rest
            off = base + nfull * KCH
            pltpu.sync_copy(col_hbm.at[pl.ds(off, rem)], idx_r)
            pltpu.sync_copy(m_hbm.at[pl.ds(off, rem)], rows_r)
            pltpu.sync_copy(rows_r, acc.at[idx_r], add=True)
        plsc.subcore_barrier()
        pltpu.sync_copy(acc.at[pl.ds(zoff, rows_per_tile)],
                        parts_hbm.at[c, pl.ds(zoff, rows_per_tile)])

    return k(m, col, zeros_tab)


def _full(shape):
    return pl.BlockSpec(shape, lambda i: (0,) * len(shape))


def _tc_prologue(at2, b2, pos3, t2, p):
    bn = 2048
    nat = p["atom_emb"].shape[0]
    nb = t2.shape[0]

    def body(at_ref, b_ref, pos_ref, t_ref, emb_w, t1w, t1b, t2w, t2b,
             h_ref, p_ref):
        tcol = t_ref[...]
        io = lax.broadcasted_iota(jnp.int32, (1, H // 2), 1).astype(_F32)
        freqs = jnp.exp(io * (-math.log(10000.0) / (H // 2 - 1)))
        args = tcol * freqs
        emb = jnp.concatenate([jnp.sin(args), jnp.cos(args)], axis=1)
        temb = _dot(_silu(_dot(emb, t1w[...]) + t1b[...]), t2w[...]) + t2b[...]
        oh_a = (at_ref[...] == lax.broadcasted_iota(jnp.int32, (bn, nat), 1)
                ).astype(_F32)
        oh_b = (b_ref[...] == lax.broadcasted_iota(jnp.int32, (bn, nb), 1)
                ).astype(_F32)
        h_ref[...] = _dot(oh_a, emb_w[...]) + _dot(oh_b, temb)
        p_ref[...] = jnp.concatenate(
            [pos_ref[...], jnp.zeros((bn, PW - 3), _F32)], axis=1)

    return pl.pallas_call(
        body,
        grid=(NPAD // bn,),
        in_specs=[
            pl.BlockSpec((bn, 1), lambda i: (i, 0)),
            pl.BlockSpec((bn, 1), lambda i: (i, 0)),
            pl.BlockSpec((bn, 3), lambda i: (i, 0)),
            _full((nb, 1)),
            _full(p["atom_emb"].shape),
            _full((H, H)), _full((1, H)),
            _full((H, H)), _full((1, H)),
        ],
        out_specs=[pl.BlockSpec((bn, H), lambda i: (i, 0)),
                   pl.BlockSpec((bn, PW), lambda i: (i, 0))],
        out_shape=[jax.ShapeDtypeStruct((NPAD, H), _F32),
                   jax.ShapeDtypeStruct((NPAD, PW), _F32)],
    )(at2, b2, pos3, t2, p["atom_emb"],
      p["time1"]["W"], p["time1"]["b"].reshape(1, H),
      p["time2"]["W"], p["time2"]["b"].reshape(1, H))


def _edge_tail(xi, xj, ea, lw):
    cat1 = jnp.concatenate([xi, xj, ea], axis=1)
    e1 = _silu(_dot(cat1, lw["e1w"][...]) + lw["e1b"][...])
    ef = _dot(e1, lw["e2w"][...]) + lw["e2b"][...]
    cat2 = jnp.concatenate([xi, xj, ef], axis=1)
    n1 = _silu(_dot(cat2, lw["n1w"][...]) + lw["n1b"][...])
    return _dot(n1, lw["n2w"][...]) + lw["n2b"][...]


def _layer_weights(lp):
    return {
        "e1w": lp["edge1"]["W"], "e1b": lp["edge1"]["b"].reshape(1, EC),
        "e2w": lp["edge2"]["W"], "e2b": lp["edge2"]["b"].reshape(1, EC),
        "n1w": lp["node1"]["W"], "n1b": lp["node1"]["b"].reshape(1, 2 * H),
        "n2w": lp["node2"]["W"], "n2b": lp["node2"]["b"].reshape(1, H),
    }


_LW_ORDER = ("e1w", "e1b", "e2w", "e2b", "n1w", "n1b", "n2w", "n2b")


def _tc_edge1(Xi, Xj, Pi, Pj, lp, p):
    e = Xi.shape[0]
    be = 2000
    lw = _layer_weights(lp)

    def body(xi_ref, xj_ref, pi_ref, pj_ref, *refs):
        lwr = dict(zip(_LW_ORDER, refs[:8]))
        ec1w, ec1b, ec2w, ec2b, m_ref, ea_ref = refs[8:]
        rel = pj_ref[...][:, :3] - pi_ref[...][:, :3]
        dist = jnp.sqrt(jnp.sum(rel * rel, axis=1, keepdims=True))
        ea0 = _silu(dist * ec1w[...] + ec1b[...])
        ea = _dot(ea0, ec2w[...]) + ec2b[...]
        ea_ref[...] = ea
        m_ref[...] = _edge_tail(xi_ref[...], xj_ref[...], ea, lwr)

    lw_vals = [lw[k] for k in _LW_ORDER]
    return pl.pallas_call(
        body,
        grid=(e // be,),
        in_specs=[pl.BlockSpec((be, H), lambda i: (i, 0))] * 2
        + [pl.BlockSpec((be, PW), lambda i: (i, 0))] * 2
        + [_full(v.shape) for v in lw_vals]
        + [_full((1, EC))] * 2 + [_full((EC, EC)), _full((1, EC))],
        out_specs=[pl.BlockSpec((be, H), lambda i: (i, 0)),
                   pl.BlockSpec((be, EC), lambda i: (i, 0))],
        out_shape=[jax.ShapeDtypeStruct((e, H), _F32),
                   jax.ShapeDtypeStruct((e, EC), _F32)],
    )(Xi, Xj, Pi, Pj, *lw_vals,
      p["edge_enc1"]["W"].reshape(1, EC), p["edge_enc1"]["b"].reshape(1, EC),
      p["edge_enc2"]["W"], p["edge_enc2"]["b"].reshape(1, EC))


def _tc_edge2(Xi, Xj, ea_arr, lp):
    e = Xi.shape[0]
    be = 2000
    lw = _layer_weights(lp)

    def body(xi_ref, xj_ref, ea_ref, *refs):
        lwr = dict(zip(_LW_ORDER, refs[:8]))
        m_ref = refs[8]
        m_ref[...] = _edge_tail(xi_ref[...], xj_ref[...], ea_ref[...], lwr)

    lw_vals = [lw[k] for k in _LW_ORDER]
    return pl.pallas_call(
        body,
        grid=(e // be,),
        in_specs=[pl.BlockSpec((be, H), lambda i: (i, 0))] * 2
        + [pl.BlockSpec((be, EC), lambda i: (i, 0))]
        + [_full(v.shape) for v in lw_vals],
        out_specs=pl.BlockSpec((be, H), lambda i: (i, 0)),
        out_shape=jax.ShapeDtypeStruct((e, H), _F32),
    )(Xi, Xj, ea_arr, *lw_vals)


def _tc_update(table, parts):
    bn = 2048

    def body(t_ref, a_ref, b_ref, o_ref):
        o_ref[...] = t_ref[...] + a_ref[...][0] + b_ref[...][0]

    return pl.pallas_call(
        body,
        grid=(NPAD // bn,),
        in_specs=[pl.BlockSpec((bn, H), lambda i: (i, 0)),
                  pl.BlockSpec((1, bn, H), lambda i: (0, i, 0)),
                  pl.BlockSpec((1, bn, H), lambda i: (1, i, 0))],
        out_specs=pl.BlockSpec((bn, H), lambda i: (i, 0)),
        out_shape=jax.ShapeDtypeStruct((NPAD, H), _F32),
    )(table, parts.reshape(2, NPAD, H), parts.reshape(2, NPAD, H))


def _tc_epilogue(table, parts, p, n):
    bn = 2000
    nat_out = p["atom2"]["W"].shape[1]

    def body(t_ref, a_ref, b_ref, wp1, bp1, wp2, bp2, wa1, ba1, wa2, ba2,
             pn_ref, al_ref):
        h = t_ref[...] + a_ref[...][0] + b_ref[...][0]
        pn_ref[...] = _dot(_silu(_dot(h, wp1[...]) + bp1[...]),
                           wp2[...]) + bp2[...]
        al_ref[...] = _dot(_silu(_dot(h, wa1[...]) + ba1[...]),
                           wa2[...]) + ba2[...]

    return pl.pallas_call(
        body,
        grid=(n // bn,),
        in_specs=[pl.BlockSpec((bn, H), lambda i: (i, 0)),
                  pl.BlockSpec((1, bn, H), lambda i: (0, i, 0)),
                  pl.BlockSpec((1, bn, H), lambda i: (1, i, 0))]
        + [_full((H, H)), _full((1, H)), _full((H, 3)), _full((1, 3)),
           _full((H, H)), _full((1, H)), _full((H, nat_out)),
           _full((1, nat_out))],
        out_specs=[pl.BlockSpec((bn, 3), lambda i: (i, 0)),
                   pl.BlockSpec((bn, nat_out), lambda i: (i, 0))],
        out_shape=[jax.ShapeDtypeStruct((n, 3), _F32),
                   jax.ShapeDtypeStruct((n, nat_out), _F32)],
    )(table, parts, parts,
      p["pos1"]["W"], p["pos1"]["b"].reshape(1, H),
      p["pos2"]["W"], p["pos2"]["b"].reshape(1, 3),
      p["atom1"]["W"], p["atom1"]["b"].reshape(1, H),
      p["atom2"]["W"], p["atom2"]["b"].reshape(1, nat_out))


def kernel(x, pos, params, atom_type, batch, t, edge_index):
    del x
    p = params
    n = pos.shape[0]
    pad = NPAD - n
    row = edge_index[0].astype(jnp.int32)
    col = edge_index[1].astype(jnp.int32)
    at2 = jnp.pad(atom_type.astype(jnp.int32), (0, pad)).reshape(NPAD, 1)
    b2 = jnp.pad(batch.astype(jnp.int32), (0, pad)).reshape(NPAD, 1)
    pos3 = jnp.pad(pos.astype(_F32), ((0, pad), (0, 0)))
    t2 = t.astype(_F32).reshape(-1, 1)
    zeros_tab = jnp.zeros((NPAD, H), _F32)

    table, pos_tab = _tc_prologue(at2, b2, pos3, t2, p)
    ea = None
    num_layers = len(p["layers"])
    for li, lp in enumerate(p["layers"]):
        if li == 0:
            Xi, Xj, Pi, Pj = _sc_gather(table, col, row, pos_tab=pos_tab)
            M, ea = _tc_edge1(Xi, Xj, Pi, Pj, lp, p)
        else:
            Xi, Xj = _sc_gather(table, col, row)
            M = _tc_edge2(Xi, Xj, ea, lp)
        parts = _sc_scatter(M, col, zeros_tab)
        if li < num_layers - 1:
            table = _tc_update(table, parts)
        else:
            out = _tc_epilogue(table, parts, p, n)
    return out


# speedup vs baseline: 2.9123x; 1.1043x over previous
import functools
import math

import jax
import jax.numpy as jnp
from jax import lax
from jax.experimental import pallas as pl
from jax.experimental.pallas import tpu as pltpu
from jax.experimental.pallas import tpu_sc as plsc

H = 128
EC = 16
PW = 128
NW = 32
KCH = 128
NPAD = 10240

_F32 = jnp.float32


def _silu(v):
    return v * (1.0 / (1.0 + jnp.exp(-v)))


def _dot(a, b):
    return jnp.dot(a, b, preferred_element_type=_F32)


def _sc_gather(table, col, row, pos_xyz=None):
    e = col.shape[0]
    per_w = e // NW
    nfull = per_w // KCH
    rem = per_w - nfull * KCH
    with_dist = pos_xyz is not None
    npad = table.shape[0]
    mesh = plsc.VectorSubcoreMesh(core_axis_name="c", subcore_axis_name="s")

    scratch = [
        pltpu.VMEM((KCH,), jnp.int32),
        pltpu.VMEM((KCH, H), _F32),
        pltpu.SemaphoreType.DMA,
    ]
    out_type = [
        jax.ShapeDtypeStruct((e, H), _F32),
        jax.ShapeDtypeStruct((e, H), _F32),
    ]
    if with_dist:
        scratch += [
            pltpu.VMEM((npad,), _F32),
            pltpu.VMEM((npad,), _F32),
            pltpu.VMEM((npad,), _F32),
            pltpu.VMEM((per_w,), jnp.int32),
            pltpu.VMEM((per_w,), jnp.int32),
            pltpu.VMEM((per_w,), _F32),
        ]
        out_type.append(jax.ShapeDtypeStruct((e,), _F32))
    if rem:
        scratch += [pltpu.VMEM((rem,), jnp.int32), pltpu.VMEM((rem, H), _F32)]

    @functools.partial(
        pl.kernel,
        mesh=mesh,
        out_type=tuple(out_type),
        scratch_types=scratch,
        compiler_params=pltpu.CompilerParams(needs_layout_passes=False),
    )
    def k(*refs):
        if with_dist:
            (table_hbm, px_hbm, py_hbm, pz_hbm, col_hbm, row_hbm,
             xi_hbm, xj_hbm, d2_hbm) = refs[:9]
            rest = refs[9:]
            idx_v, rows_v, sem, px_t, py_t, pz_t, ic_v, ir_v, d2_v = rest[:9]
            rest = rest[9:]
        else:
            table_hbm, col_hbm, row_hbm, xi_hbm, xj_hbm = refs[:5]
            rest = refs[5:]
            idx_v, rows_v, sem = rest[:3]
            rest = rest[3:]
        wid = lax.axis_index("s") * 2 + lax.axis_index("c")
        base = wid * per_w

        if with_dist:
            pltpu.sync_copy(px_hbm, px_t)
            pltpu.sync_copy(py_hbm, py_t)
            pltpu.sync_copy(pz_hbm, pz_t)
            pltpu.sync_copy(col_hbm.at[pl.ds(base, per_w)], ic_v)
            pltpu.sync_copy(row_hbm.at[pl.ds(base, per_w)], ir_v)

            def d2_at(s):
                ic = ic_v[pl.ds(s, 16)]
                ir = ir_v[pl.ds(s, 16)]
                dx = plsc.load_gather(px_t, [ir]) - plsc.load_gather(px_t, [ic])
                dy = plsc.load_gather(py_t, [ir]) - plsc.load_gather(py_t, [ic])
                dz = plsc.load_gather(pz_t, [ir]) - plsc.load_gather(pz_t, [ic])
                d2_v[pl.ds(s, 16)] = dx * dx + dy * dy + dz * dz

            def dbody(kk, _):
                d2_at(kk * 16)
                return 0
            lax.fori_loop(0, per_w // 16, dbody, 0)
            if per_w % 16:
                d2_at(per_w - 16)
            pltpu.sync_copy(d2_v, d2_hbm.at[pl.ds(base, per_w)])

        for idx_hbm, out_hbm in ((col_hbm, xi_hbm), (row_hbm, xj_hbm)):
            def body(ci, _, idx_hbm=idx_hbm, out_hbm=out_hbm):
                off = base + ci * KCH
                pltpu.sync_copy(idx_hbm.at[pl.ds(off, KCH)], idx_v)
                pltpu.async_copy(table_hbm.at[idx_v], rows_v, sem).wait()
                pltpu.sync_copy(rows_v, out_hbm.at[pl.ds(off, KCH)])
                return 0
            lax.fori_loop(0, nfull, body, 0)
            if rem:
                idx_r, rows_r = rest
                off = base + nfull * KCH
                pltpu.sync_copy(idx_hbm.at[pl.ds(off, rem)], idx_r)
                pltpu.async_copy(table_hbm.at[idx_r], rows_r, sem).wait()
                pltpu.sync_copy(rows_r, out_hbm.at[pl.ds(off, rem)])

    if with_dist:
        return k(table, pos_xyz[0], pos_xyz[1], pos_xyz[2], col, row)
    return k(table, col, row)


def _sc_scatter(m, col, zeros_tab):
    e = col.shape[0]
    per_w = e // NW
    nfull = per_w // KCH
    rem = per_w - nfull * KCH
    rows_per_tile = NPAD // 16
    mesh = plsc.VectorSubcoreMesh(core_axis_name="c", subcore_axis_name="s")

    scratch = [
        pltpu.VMEM((KCH,), jnp.int32),
        pltpu.VMEM((KCH, H), _F32),
        pltpu.VMEM_SHARED((NPAD, H), _F32),
        pltpu.SemaphoreType.DMA,
    ]
    if rem:
        scratch += [pltpu.VMEM((rem,), jnp.int32), pltpu.VMEM((rem, H), _F32)]

    @functools.partial(
        pl.kernel,
        mesh=mesh,
        out_type=jax.ShapeDtypeStruct((2, NPAD, H), _F32),
        scratch_types=scratch,
    )
    def k(m_hbm, col_hbm, zeros_hbm, parts_hbm, idx_v, rows_v, acc, sem,
          *rest):
        c = lax.axis_index("c")
        s = lax.axis_index("s")
        wid = s * 2 + c
        base = wid * per_w
        zoff = s * rows_per_tile
        pltpu.sync_copy(zeros_hbm.at[pl.ds(zoff, rows_per_tile)],
                        acc.at[pl.ds(zoff, rows_per_tile)])
        plsc.subcore_barrier()

        def body(ci, _):
            off = base + ci * KCH
            pltpu.sync_copy(col_hbm.at[pl.ds(off, KCH)], idx_v)
            pltpu.sync_copy(m_hbm.at[pl.ds(off, KCH)], rows_v)
            pltpu.sync_copy(rows_v, acc.at[idx_v], add=True)
            return 0
        lax.fori_loop(0, nfull, body, 0)
        if rem:
            idx_r, rows_r = rest
            off = base + nfull * KCH
            pltpu.sync_copy(col_hbm.at[pl.ds(off, rem)], idx_r)
            pltpu.sync_copy(m_hbm.at[pl.ds(off, rem)], rows_r)
            pltpu.sync_copy(rows_r, acc.at[idx_r], add=True)
        plsc.subcore_barrier()
        pltpu.sync_copy(acc.at[pl.ds(zoff, rows_per_tile)],
                        parts_hbm.at[c, pl.ds(zoff, rows_per_tile)])

    return k(m, col, zeros_tab)


def _full(shape):
    return pl.BlockSpec(shape, lambda i: (0,) * len(shape))


def _tc_prologue(at2, b2, t2, p):
    bn = 2048
    nat = p["atom_emb"].shape[0]
    nb = t2.shape[0]

    def body(at_ref, b_ref, t_ref, emb_w, t1w, t1b, t2w, t2b, h_ref):
        tcol = t_ref[...]
        io = lax.broadcasted_iota(jnp.int32, (1, H // 2), 1).astype(_F32)
        freqs = jnp.exp(io * (-math.log(10000.0) / (H // 2 - 1)))
        args = tcol * freqs
        emb = jnp.concatenate([jnp.sin(args), jnp.cos(args)], axis=1)
        temb = _dot(_silu(_dot(emb, t1w[...]) + t1b[...]), t2w[...]) + t2b[...]
        oh_a = (at_ref[...] == lax.broadcasted_iota(jnp.int32, (bn, nat), 1)
                ).astype(_F32)
        oh_b = (b_ref[...] == lax.broadcasted_iota(jnp.int32, (bn, nb), 1)
                ).astype(_F32)
        h_ref[...] = _dot(oh_a, emb_w[...]) + _dot(oh_b, temb)

    return pl.pallas_call(
        body,
        grid=(NPAD // bn,),
        in_specs=[
            pl.BlockSpec((bn, 1), lambda i: (i, 0)),
            pl.BlockSpec((bn, 1), lambda i: (i, 0)),
            _full((nb, 1)),
            _full(p["atom_emb"].shape),
            _full((H, H)), _full((1, H)),
            _full((H, H)), _full((1, H)),
        ],
        out_specs=pl.BlockSpec((bn, H), lambda i: (i, 0)),
        out_shape=jax.ShapeDtypeStruct((NPAD, H), _F32),
    )(at2, b2, t2, p["atom_emb"],
      p["time1"]["W"], p["time1"]["b"].reshape(1, H),
      p["time2"]["W"], p["time2"]["b"].reshape(1, H))


def _edge_tail(xi, xj, ea, lw):
    cat1 = jnp.concatenate([xi, xj, ea], axis=1)
    e1 = _silu(_dot(cat1, lw["e1w"][...]) + lw["e1b"][...])
    ef = _dot(e1, lw["e2w"][...]) + lw["e2b"][...]
    cat2 = jnp.concatenate([xi, xj, ef], axis=1)
    n1 = _silu(_dot(cat2, lw["n1w"][...]) + lw["n1b"][...])
    return _dot(n1, lw["n2w"][...]) + lw["n2b"][...]


def _layer_weights(lp):
    return {
        "e1w": lp["edge1"]["W"], "e1b": lp["edge1"]["b"].reshape(1, EC),
        "e2w": lp["edge2"]["W"], "e2b": lp["edge2"]["b"].reshape(1, EC),
        "n1w": lp["node1"]["W"], "n1b": lp["node1"]["b"].reshape(1, 2 * H),
        "n2w": lp["node2"]["W"], "n2b": lp["node2"]["b"].reshape(1, H),
    }


_LW_ORDER = ("e1w", "e1b", "e2w", "e2b", "n1w", "n1b", "n2w", "n2b")


def _tc_edge1(Xi, Xj, d2, lp, p):
    e = Xi.shape[0]
    be = 2000
    lw = _layer_weights(lp)

    def body(xi_ref, xj_ref, d2_ref, *refs):
        lwr = dict(zip(_LW_ORDER, refs[:8]))
        ec1w, ec1b, ec2w, ec2b, m_ref, ea_ref = refs[8:]
        dist = jnp.sqrt(d2_ref[...])
        ea0 = _silu(dist * ec1w[...] + ec1b[...])
        ea = _dot(ea0, ec2w[...]) + ec2b[...]
        ea_ref[...] = ea
        m_ref[...] = _edge_tail(xi_ref[...], xj_ref[...], ea, lwr)

    lw_vals = [lw[k] for k in _LW_ORDER]
    return pl.pallas_call(
        body,
        grid=(e // be,),
        in_specs=[pl.BlockSpec((be, H), lambda i: (i, 0))] * 2
        + [pl.BlockSpec((be, 1), lambda i: (i, 0))]
        + [_full(v.shape) for v in lw_vals]
        + [_full((1, EC))] * 2 + [_full((EC, EC)), _full((1, EC))],
        out_specs=[pl.BlockSpec((be, H), lambda i: (i, 0)),
                   pl.BlockSpec((be, EC), lambda i: (i, 0))],
        out_shape=[jax.ShapeDtypeStruct((e, H), _F32),
                   jax.ShapeDtypeStruct((e, EC), _F32)],
    )(Xi, Xj, d2, *lw_vals,
      p["edge_enc1"]["W"].reshape(1, EC), p["edge_enc1"]["b"].reshape(1, EC),
      p["edge_enc2"]["W"], p["edge_enc2"]["b"].reshape(1, EC))


def _tc_edge2(Xi, Xj, ea_arr, lp):
    e = Xi.shape[0]
    be = 2000
    lw = _layer_weights(lp)

    def body(xi_ref, xj_ref, ea_ref, *refs):
        lwr = dict(zip(_LW_ORDER, refs[:8]))
        m_ref = refs[8]
        m_ref[...] = _edge_tail(xi_ref[...], xj_ref[...], ea_ref[...], lwr)

    lw_vals = [lw[k] for k in _LW_ORDER]
    return pl.pallas_call(
        body,
        grid=(e // be,),
        in_specs=[pl.BlockSpec((be, H), lambda i: (i, 0))] * 2
        + [pl.BlockSpec((be, EC), lambda i: (i, 0))]
        + [_full(v.shape) for v in lw_vals],
        out_specs=pl.BlockSpec((be, H), lambda i: (i, 0)),
        out_shape=jax.ShapeDtypeStruct((e, H), _F32),
    )(Xi, Xj, ea_arr, *lw_vals)


def _tc_update(table, parts):
    bn = 2048

    def body(t_ref, a_ref, b_ref, o_ref):
        o_ref[...] = t_ref[...] + a_ref[...][0] + b_ref[...][0]

    return pl.pallas_call(
        body,
        grid=(NPAD // bn,),
        in_specs=[pl.BlockSpec((bn, H), lambda i: (i, 0)),
                  pl.BlockSpec((1, bn, H), lambda i: (0, i, 0)),
                  pl.BlockSpec((1, bn, H), lambda i: (1, i, 0))],
        out_specs=pl.BlockSpec((bn, H), lambda i: (i, 0)),
        out_shape=jax.ShapeDtypeStruct((NPAD, H), _F32),
    )(table, parts.reshape(2, NPAD, H), parts.reshape(2, NPAD, H))


def _tc_epilogue(table, parts, p, n):
    bn = 2000
    nat_out = p["atom2"]["W"].shape[1]

    def body(t_ref, a_ref, b_ref, wp1, bp1, wp2, bp2, wa1, ba1, wa2, ba2,
             pn_ref, al_ref):
        h = t_ref[...] + a_ref[...][0] + b_ref[...][0]
        pn_ref[...] = _dot(_silu(_dot(h, wp1[...]) + bp1[...]),
                           wp2[...]) + bp2[...]
        al_ref[...] = _dot(_silu(_dot(h, wa1[...]) + ba1[...]),
                           wa2[...]) + ba2[...]

    return pl.pallas_call(
        body,
        grid=(n // bn,),
        in_specs=[pl.BlockSpec((bn, H), lambda i: (i, 0)),
                  pl.BlockSpec((1, bn, H), lambda i: (0, i, 0)),
                  pl.BlockSpec((1, bn, H), lambda i: (1, i, 0))]
        + [_full((H, H)), _full((1, H)), _full((H, 3)), _full((1, 3)),
           _full((H, H)), _full((1, H)), _full((H, nat_out)),
           _full((1, nat_out))],
        out_specs=[pl.BlockSpec((bn, 3), lambda i: (i, 0)),
                   pl.BlockSpec((bn, nat_out), lambda i: (i, 0))],
        out_shape=[jax.ShapeDtypeStruct((n, 3), _F32),
                   jax.ShapeDtypeStruct((n, nat_out), _F32)],
    )(table, parts, parts,
      p["pos1"]["W"], p["pos1"]["b"].reshape(1, H),
      p["pos2"]["W"], p["pos2"]["b"].reshape(1, 3),
      p["atom1"]["W"], p["atom1"]["b"].reshape(1, H),
      p["atom2"]["W"], p["atom2"]["b"].reshape(1, nat_out))


def kernel(x, pos, params, atom_type, batch, t, edge_index):
    del x
    p = params
    n = pos.shape[0]
    pad = NPAD - n
    row = edge_index[0].astype(jnp.int32)
    col = edge_index[1].astype(jnp.int32)
    at2 = jnp.pad(atom_type.astype(jnp.int32), (0, pad)).reshape(NPAD, 1)
    b2 = jnp.pad(batch.astype(jnp.int32), (0, pad)).reshape(NPAD, 1)
    pos3 = jnp.pad(pos.astype(_F32), ((0, pad), (0, 0)))
    pos_xyz = (pos3[:, 0], pos3[:, 1], pos3[:, 2])
    t2 = t.astype(_F32).reshape(-1, 1)
    zeros_tab = jnp.zeros((NPAD, H), _F32)

    table = _tc_prologue(at2, b2, t2, p)
    ea = None
    num_layers = len(p["layers"])
    for li, lp in enumerate(p["layers"]):
        if li == 0:
            Xi, Xj, d2 = _sc_gather(table, col, row, pos_xyz=pos_xyz)
            M, ea = _tc_edge1(Xi, Xj, d2.reshape(-1, 1), lp, p)
        else:
            Xi, Xj = _sc_gather(table, col, row)
            M = _tc_edge2(Xi, Xj, ea, lp)
        parts = _sc_scatter(M, col, zeros_tab)
        if li < num_layers - 1:
            table = _tc_update(table, parts)
        else:
            out = _tc_epilogue(table, parts, p, n)
    return out


# speedup vs baseline: 3.5878x; 1.2320x over previous
import functools
import math

import jax
import jax.numpy as jnp
from jax import lax
from jax.experimental import pallas as pl
from jax.experimental.pallas import tpu as pltpu
from jax.experimental.pallas import tpu_sc as plsc

H = 128
EC = 16
PW = 128
NW = 32
KCH = 128
NPAD = 10240

_F32 = jnp.float32


def _silu(v):
    return v * (1.0 / (1.0 + jnp.exp(-v)))


def _dot(a, b):
    return jnp.dot(a, b, preferred_element_type=_F32)


def _sc_gather(table, col, row, pos_xyz=None):
    e = col.shape[0]
    per_w = e // NW
    nfull = per_w // KCH
    rem = per_w - nfull * KCH
    with_dist = pos_xyz is not None
    npad = table.shape[0]
    mesh = plsc.VectorSubcoreMesh(core_axis_name="c", subcore_axis_name="s")

    scratch = [
        pltpu.VMEM((KCH,), jnp.int32),
        pltpu.VMEM((KCH, H), _F32),
        pltpu.SemaphoreType.DMA,
    ]
    out_type = [
        jax.ShapeDtypeStruct((e, H), _F32),
        jax.ShapeDtypeStruct((e, H), _F32),
    ]
    if with_dist:
        scratch += [
            pltpu.VMEM((npad,), _F32),
            pltpu.VMEM((npad,), _F32),
            pltpu.VMEM((npad,), _F32),
            pltpu.VMEM((per_w,), jnp.int32),
            pltpu.VMEM((per_w,), jnp.int32),
            pltpu.VMEM((per_w,), _F32),
        ]
        out_type.append(jax.ShapeDtypeStruct((e,), _F32))
    if rem:
        scratch += [pltpu.VMEM((rem,), jnp.int32), pltpu.VMEM((rem, H), _F32)]

    @functools.partial(
        pl.kernel,
        mesh=mesh,
        out_type=tuple(out_type),
        scratch_types=scratch,
        compiler_params=pltpu.CompilerParams(needs_layout_passes=False),
    )
    def k(*refs):
        if with_dist:
            (table_hbm, px_hbm, py_hbm, pz_hbm, col_hbm, row_hbm,
             xi_hbm, xj_hbm, d2_hbm) = refs[:9]
            rest = refs[9:]
            idx_v, rows_v, sem, px_t, py_t, pz_t, ic_v, ir_v, d2_v = rest[:9]
            rest = rest[9:]
        else:
            table_hbm, col_hbm, row_hbm, xi_hbm, xj_hbm = refs[:5]
            rest = refs[5:]
            idx_v, rows_v, sem = rest[:3]
            rest = rest[3:]
        wid = lax.axis_index("s") * 2 + lax.axis_index("c")
        base = wid * per_w

        if with_dist:
            pltpu.sync_copy(px_hbm, px_t)
            pltpu.sync_copy(py_hbm, py_t)
            pltpu.sync_copy(pz_hbm, pz_t)
            pltpu.sync_copy(col_hbm.at[pl.ds(base, per_w)], ic_v)
            pltpu.sync_copy(row_hbm.at[pl.ds(base, per_w)], ir_v)

            def d2_at(s):
                ic = ic_v[pl.ds(s, 16)]
                ir = ir_v[pl.ds(s, 16)]
                dx = plsc.load_gather(px_t, [ir]) - plsc.load_gather(px_t, [ic])
                dy = plsc.load_gather(py_t, [ir]) - plsc.load_gather(py_t, [ic])
                dz = plsc.load_gather(pz_t, [ir]) - plsc.load_gather(pz_t, [ic])
                d2_v[pl.ds(s, 16)] = dx * dx + dy * dy + dz * dz

            def dbody(kk, _):
                d2_at(kk * 16)
                return 0
            lax.fori_loop(0, per_w // 16, dbody, 0)
            if per_w % 16:
                d2_at(per_w - 16)
            pltpu.sync_copy(d2_v, d2_hbm.at[pl.ds(base, per_w)])

        for idx_hbm, out_hbm in ((col_hbm, xi_hbm), (row_hbm, xj_hbm)):
            def body(ci, _, idx_hbm=idx_hbm, out_hbm=out_hbm):
                off = base + ci * KCH
                pltpu.sync_copy(idx_hbm.at[pl.ds(off, KCH)], idx_v)
                pltpu.async_copy(table_hbm.at[idx_v], rows_v, sem).wait()
                pltpu.sync_copy(rows_v, out_hbm.at[pl.ds(off, KCH)])
                return 0
            lax.fori_loop(0, nfull, body, 0)
            if rem:
                idx_r, rows_r = rest
                off = base + nfull * KCH
                pltpu.sync_copy(idx_hbm.at[pl.ds(off, rem)], idx_r)
                pltpu.async_copy(table_hbm.at[idx_r], rows_r, sem).wait()
                pltpu.sync_copy(rows_r, out_hbm.at[pl.ds(off, rem)])

    if with_dist:
        return k(table, pos_xyz[0], pos_xyz[1], pos_xyz[2], col, row)
    return k(table, col, row)


def _sc_scatter(m, col, zeros_tab):
    e = col.shape[0]
    per_w = e // NW
    nfull = per_w // KCH
    rem = per_w - nfull * KCH
    rows_per_tile = NPAD // 16
    mesh = plsc.VectorSubcoreMesh(core_axis_name="c", subcore_axis_name="s")

    scratch = [
        pltpu.VMEM((KCH,), jnp.int32),
        pltpu.VMEM((KCH, H), _F32),
        pltpu.VMEM_SHARED((NPAD, H), _F32),
        pltpu.SemaphoreType.DMA,
    ]
    if rem:
        scratch += [pltpu.VMEM((rem,), jnp.int32), pltpu.VMEM((rem, H), _F32)]

    @functools.partial(
        pl.kernel,
        mesh=mesh,
        out_type=jax.ShapeDtypeStruct((2, NPAD, H), _F32),
        scratch_types=scratch,
    )
    def k(m_hbm, col_hbm, zeros_hbm, parts_hbm, idx_v, rows_v, acc, sem,
          *rest):
        c = lax.axis_index("c")
        s = lax.axis_index("s")
        wid = s * 2 + c
        base = wid * per_w
        zoff = s * rows_per_tile
        pltpu.sync_copy(zeros_hbm.at[pl.ds(zoff, rows_per_tile)],
                        acc.at[pl.ds(zoff, rows_per_tile)])
        plsc.subcore_barrier()

        def body(ci, _):
            off = base + ci * KCH
            pltpu.sync_copy(col_hbm.at[pl.ds(off, KCH)], idx_v)
            pltpu.sync_copy(m_hbm.at[pl.ds(off, KCH)], rows_v)
            pltpu.sync_copy(rows_v, acc.at[idx_v], add=True)
            return 0
        lax.fori_loop(0, nfull, body, 0)
        if rem:
            idx_r, rows_r = rest
            off = base + nfull * KCH
            pltpu.sync_copy(col_hbm.at[pl.ds(off, rem)], idx_r)
            pltpu.sync_copy(m_hbm.at[pl.ds(off, rem)], rows_r)
            pltpu.sync_copy(rows_r, acc.at[idx_r], add=True)
        plsc.subcore_barrier()
        pltpu.sync_copy(acc.at[pl.ds(zoff, rows_per_tile)],
                        parts_hbm.at[c, pl.ds(zoff, rows_per_tile)])

    return k(m, col, zeros_tab)


def _full(shape):
    return pl.BlockSpec(shape, lambda i: (0,) * len(shape))


def _tc_prologue(at2, b2, t2, p):
    bn = 2048
    nat = p["atom_emb"].shape[0]
    nb = t2.shape[0]

    def body(at_ref, b_ref, t_ref, emb_w, t1w, t1b, t2w, t2b, h_ref):
        tcol = t_ref[...]
        io = lax.broadcasted_iota(jnp.int32, (1, H // 2), 1).astype(_F32)
        freqs = jnp.exp(io * (-math.log(10000.0) / (H // 2 - 1)))
        args = tcol * freqs
        emb = jnp.concatenate([jnp.sin(args), jnp.cos(args)], axis=1)
        temb = _dot(_silu(_dot(emb, t1w[...]) + t1b[...]), t2w[...]) + t2b[...]
        oh_a = (at_ref[...] == lax.broadcasted_iota(jnp.int32, (bn, nat), 1)
                ).astype(_F32)
        oh_b = (b_ref[...] == lax.broadcasted_iota(jnp.int32, (bn, nb), 1)
                ).astype(_F32)
        h_ref[...] = _dot(oh_a, emb_w[...]) + _dot(oh_b, temb)

    return pl.pallas_call(
        body,
        grid=(NPAD // bn,),
        in_specs=[
            pl.BlockSpec((bn, 1), lambda i: (i, 0)),
            pl.BlockSpec((bn, 1), lambda i: (i, 0)),
            _full((nb, 1)),
            _full(p["atom_emb"].shape),
            _full((H, H)), _full((1, H)),
            _full((H, H)), _full((1, H)),
        ],
        out_specs=pl.BlockSpec((bn, H), lambda i: (i, 0)),
        out_shape=jax.ShapeDtypeStruct((NPAD, H), _F32),
    )(at2, b2, t2, p["atom_emb"],
      p["time1"]["W"], p["time1"]["b"].reshape(1, H),
      p["time2"]["W"], p["time2"]["b"].reshape(1, H))


def _edge_tail(xi, xj, ea, lw):
    cat1 = jnp.concatenate([xi, xj, ea], axis=1)
    e1 = _silu(_dot(cat1, lw["e1w"][...]) + lw["e1b"][...])
    ef = _dot(e1, lw["e2w"][...]) + lw["e2b"][...]
    cat2 = jnp.concatenate([xi, xj, ef], axis=1)
    n1 = _silu(_dot(cat2, lw["n1w"][...]) + lw["n1b"][...])
    return _dot(n1, lw["n2w"][...]) + lw["n2b"][...]


def _layer_weights(lp):
    return {
        "e1w": lp["edge1"]["W"], "e1b": lp["edge1"]["b"].reshape(1, EC),
        "e2w": lp["edge2"]["W"], "e2b": lp["edge2"]["b"].reshape(1, EC),
        "n1w": lp["node1"]["W"], "n1b": lp["node1"]["b"].reshape(1, 2 * H),
        "n2w": lp["node2"]["W"], "n2b": lp["node2"]["b"].reshape(1, H),
    }


_LW_ORDER = ("e1w", "e1b", "e2w", "e2b", "n1w", "n1b", "n2w", "n2b")


def _edge_block(e):
    return 2048 if e % 2048 == 0 else e // 32


def _tc_edge1(Xi, Xj, d2, lp, p):
    e = Xi.shape[0]
    be = _edge_block(e)
    lw = _layer_weights(lp)

    def body(xi_ref, xj_ref, d2_ref, *refs):
        lwr = dict(zip(_LW_ORDER, refs[:8]))
        ec1w, ec1b, ec2w, ec2b, m_ref, ea_ref = refs[8:]
        dist = jnp.sqrt(d2_ref[...])
        ea0 = _silu(dist * ec1w[...] + ec1b[...])
        ea = _dot(ea0, ec2w[...]) + ec2b[...]
        ea_ref[...] = ea
        m_ref[...] = _edge_tail(xi_ref[...], xj_ref[...], ea, lwr)

    lw_vals = [lw[k] for k in _LW_ORDER]
    return pl.pallas_call(
        body,
        grid=(e // be,),
        in_specs=[pl.BlockSpec((be, H), lambda i: (i, 0))] * 2
        + [pl.BlockSpec((be, 1), lambda i: (i, 0))]
        + [_full(v.shape) for v in lw_vals]
        + [_full((1, EC))] * 2 + [_full((EC, EC)), _full((1, EC))],
        out_specs=[pl.BlockSpec((be, H), lambda i: (i, 0)),
                   pl.BlockSpec((be, EC), lambda i: (i, 0))],
        out_shape=[jax.ShapeDtypeStruct((e, H), _F32),
                   jax.ShapeDtypeStruct((e, EC), _F32)],
    )(Xi, Xj, d2, *lw_vals,
      p["edge_enc1"]["W"].reshape(1, EC), p["edge_enc1"]["b"].reshape(1, EC),
      p["edge_enc2"]["W"], p["edge_enc2"]["b"].reshape(1, EC))


def _tc_edge2(Xi, Xj, ea_arr, lp):
    e = Xi.shape[0]
    be = _edge_block(e)
    lw = _layer_weights(lp)

    def body(xi_ref, xj_ref, ea_ref, *refs):
        lwr = dict(zip(_LW_ORDER, refs[:8]))
        m_ref = refs[8]
        m_ref[...] = _edge_tail(xi_ref[...], xj_ref[...], ea_ref[...], lwr)

    lw_vals = [lw[k] for k in _LW_ORDER]
    return pl.pallas_call(
        body,
        grid=(e // be,),
        in_specs=[pl.BlockSpec((be, H), lambda i: (i, 0))] * 2
        + [pl.BlockSpec((be, EC), lambda i: (i, 0))]
        + [_full(v.shape) for v in lw_vals],
        out_specs=pl.BlockSpec((be, H), lambda i: (i, 0)),
        out_shape=jax.ShapeDtypeStruct((e, H), _F32),
    )(Xi, Xj, ea_arr, *lw_vals)


def _tc_update(table, parts_a, parts_b):
    bn = 2048

    def body(t_ref, a0, a1, b0, b1, o_ref):
        o_ref[...] = (t_ref[...] + a0[...][0] + a1[...][0]
                      + b0[...][0] + b1[...][0])

    return pl.pallas_call(
        body,
        grid=(NPAD // bn,),
        in_specs=[pl.BlockSpec((bn, H), lambda i: (i, 0)),
                  pl.BlockSpec((1, bn, H), lambda i: (0, i, 0)),
                  pl.BlockSpec((1, bn, H), lambda i: (1, i, 0)),
                  pl.BlockSpec((1, bn, H), lambda i: (0, i, 0)),
                  pl.BlockSpec((1, bn, H), lambda i: (1, i, 0))],
        out_specs=pl.BlockSpec((bn, H), lambda i: (i, 0)),
        out_shape=jax.ShapeDtypeStruct((NPAD, H), _F32),
    )(table, parts_a, parts_a, parts_b, parts_b)


def _tc_epilogue(table, parts_a, parts_b, p, n):
    bn = 2000
    nat_out = p["atom2"]["W"].shape[1]

    def body(t_ref, a0, a1, b0, b1, wp1, bp1, wp2, bp2, wa1, ba1, wa2, ba2,
             pn_ref, al_ref):
        h = (t_ref[...] + a0[...][0] + a1[...][0]
             + b0[...][0] + b1[...][0])
        pn_ref[...] = _dot(_silu(_dot(h, wp1[...]) + bp1[...]),
                           wp2[...]) + bp2[...]
        al_ref[...] = _dot(_silu(_dot(h, wa1[...]) + ba1[...]),
                           wa2[...]) + ba2[...]

    return pl.pallas_call(
        body,
        grid=(n // bn,),
        in_specs=[pl.BlockSpec((bn, H), lambda i: (i, 0)),
                  pl.BlockSpec((1, bn, H), lambda i: (0, i, 0)),
                  pl.BlockSpec((1, bn, H), lambda i: (1, i, 0)),
                  pl.BlockSpec((1, bn, H), lambda i: (0, i, 0)),
                  pl.BlockSpec((1, bn, H), lambda i: (1, i, 0))]
        + [_full((H, H)), _full((1, H)), _full((H, 3)), _full((1, 3)),
           _full((H, H)), _full((1, H)), _full((H, nat_out)),
           _full((1, nat_out))],
        out_specs=[pl.BlockSpec((bn, 3), lambda i: (i, 0)),
                   pl.BlockSpec((bn, nat_out), lambda i: (i, 0))],
        out_shape=[jax.ShapeDtypeStruct((n, 3), _F32),
                   jax.ShapeDtypeStruct((n, nat_out), _F32)],
    )(table, parts_a, parts_a, parts_b, parts_b,
      p["pos1"]["W"], p["pos1"]["b"].reshape(1, H),
      p["pos2"]["W"], p["pos2"]["b"].reshape(1, 3),
      p["atom1"]["W"], p["atom1"]["b"].reshape(1, H),
      p["atom2"]["W"], p["atom2"]["b"].reshape(1, nat_out))


def kernel(x, pos, params, atom_type, batch, t, edge_index):
    del x
    p = params
    n = pos.shape[0]
    pad = NPAD - n
    row = edge_index[0].astype(jnp.int32)
    col = edge_index[1].astype(jnp.int32)
    at2 = jnp.pad(atom_type.astype(jnp.int32), (0, pad)).reshape(NPAD, 1)
    b2 = jnp.pad(batch.astype(jnp.int32), (0, pad)).reshape(NPAD, 1)
    pos3 = jnp.pad(pos.astype(_F32), ((0, pad), (0, 0)))
    pos_xyz = (pos3[:, 0], pos3[:, 1], pos3[:, 2])
    t2 = t.astype(_F32).reshape(-1, 1)
    zeros_tab = jnp.zeros((NPAD, H), _F32)

    table = _tc_prologue(at2, b2, t2, p)

    e = col.shape[0]
    ea_split = (e // 2) // 256 * 256
    halves = [(col[:ea_split], row[:ea_split]),
              (col[ea_split:], row[ea_split:])]

    ea = [None, None]
    num_layers = len(p["layers"])
    for li, lp in enumerate(p["layers"]):
        parts = []
        for hi, (colh, rowh) in enumerate(halves):
            if li == 0:
                Xi, Xj, d2 = _sc_gather(table, colh, rowh, pos_xyz=pos_xyz)
                M, ea[hi] = _tc_edge1(Xi, Xj, d2.reshape(-1, 1), lp, p)
            else:
                Xi, Xj = _sc_gather(table, colh, rowh)
                M = _tc_edge2(Xi, Xj, ea[hi], lp)
            parts.append(_sc_scatter(M, colh, zeros_tab))
        if li < num_layers - 1:
            table = _tc_update(table, parts[0], parts[1])
        else:
            out = _tc_epilogue(table, parts[0], parts[1], p, n)
    return out


# speedup vs baseline: 3.9292x; 1.0952x over previous
import functools
import math

import jax
import jax.numpy as jnp
from jax import lax
from jax.experimental import pallas as pl
from jax.experimental.pallas import tpu as pltpu
from jax.experimental.pallas import tpu_sc as plsc

H = 128
EC = 16
PW = 128
NW = 32
KCH = 128
NPAD = 10240

_F32 = jnp.float32


def _silu(v):
    return v * (1.0 / (1.0 + jnp.exp(-v)))


def _dot(a, b):
    return jnp.dot(a, b, preferred_element_type=_F32)


def _sc_gather(table, col, row, pos_xyz=None):
    e = col.shape[0]
    per_w = e // NW
    nfull = per_w // KCH
    rem = per_w - nfull * KCH
    nb = 4
    ngrp = nfull // nb
    ltail = nfull - ngrp * nb
    with_dist = pos_xyz is not None
    npad = table.shape[0]
    mesh = plsc.VectorSubcoreMesh(core_axis_name="c", subcore_axis_name="s")

    scratch = [
        pltpu.VMEM((per_w,), jnp.int32),
        pltpu.VMEM((per_w,), jnp.int32),
    ] + [pltpu.VMEM((KCH, H), _F32) for _ in range(nb)] + [
        pltpu.SemaphoreType.DMA,
    ]
    out_type = [
        jax.ShapeDtypeStruct((e, H), _F32),
        jax.ShapeDtypeStruct((e, H), _F32),
    ]
    if rem:
        scratch.append(pltpu.VMEM((rem, H), _F32))
    if with_dist:
        scratch += [
            pltpu.VMEM((npad,), _F32),
            pltpu.VMEM((npad,), _F32),
            pltpu.VMEM((npad,), _F32),
            pltpu.VMEM((per_w,), _F32),
        ]
        out_type.append(jax.ShapeDtypeStruct((e,), _F32))

    @functools.partial(
        pl.kernel,
        mesh=mesh,
        out_type=tuple(out_type),
        scratch_types=scratch,
        compiler_params=pltpu.CompilerParams(needs_layout_passes=False),
    )
    def k(*refs):
        if with_dist:
            (table_hbm, px_hbm, py_hbm, pz_hbm, col_hbm, row_hbm,
             xi_hbm, xj_hbm, d2_hbm) = refs[:9]
            rest = refs[9:]
        else:
            table_hbm, col_hbm, row_hbm, xi_hbm, xj_hbm = refs[:5]
            rest = refs[5:]
        ic_v, ir_v = rest[:2]
        rows = rest[2:2 + nb]
        sem = rest[2 + nb]
        rest = rest[3 + nb:]
        if rem:
            rows_r = rest[0]
            rest = rest[1:]
        if with_dist:
            px_t, py_t, pz_t, d2_v = rest

        wid = lax.axis_index("s") * 2 + lax.axis_index("c")
        base = wid * per_w
        pltpu.sync_copy(col_hbm.at[pl.ds(base, per_w)], ic_v)
        pltpu.sync_copy(row_hbm.at[pl.ds(base, per_w)], ir_v)

        if with_dist:
            pltpu.sync_copy(px_hbm, px_t)
            pltpu.sync_copy(py_hbm, py_t)
            pltpu.sync_copy(pz_hbm, pz_t)

            def d2_at(s):
                ic = ic_v[pl.ds(s, 16)]
                ir = ir_v[pl.ds(s, 16)]
                dx = plsc.load_gather(px_t, [ir]) - plsc.load_gather(px_t, [ic])
                dy = plsc.load_gather(py_t, [ir]) - plsc.load_gather(py_t, [ic])
                dz = plsc.load_gather(pz_t, [ir]) - plsc.load_gather(pz_t, [ic])
                d2_v[pl.ds(s, 16)] = dx * dx + dy * dy + dz * dz

            def dbody(kk, _):
                d2_at(kk * 16)
                return 0
            lax.fori_loop(0, per_w // 16, dbody, 0)
            if per_w % 16:
                d2_at(per_w - 16)
            pltpu.sync_copy(d2_v, d2_hbm.at[pl.ds(base, per_w)])

        for idx_big, out_hbm in ((ic_v, xi_hbm), (ir_v, xj_hbm)):
            def grp(g, _, idx_big=idx_big, out_hbm=out_hbm):
                hs = [
                    pltpu.async_copy(
                        table_hbm.at[idx_big.at[pl.ds((g * nb + j) * KCH, KCH)]],
                        rows[j], sem)
                    for j in range(nb)
                ]
                for j, h in enumerate(hs):
                    h.wait()
                    pltpu.sync_copy(
                        rows[j],
                        out_hbm.at[pl.ds(base + (g * nb + j) * KCH, KCH)])
                return 0
            lax.fori_loop(0, ngrp, grp, 0)
            hs = []
            offs = []
            for j in range(ltail):
                ci = ngrp * nb + j
                hs.append(pltpu.async_copy(
                    table_hbm.at[idx_big.at[pl.ds(ci * KCH, KCH)]],
                    rows[j], sem))
                offs.append((rows[j], ci * KCH, KCH))
            if rem:
                hs.append(pltpu.async_copy(
                    table_hbm.at[idx_big.at[pl.ds(nfull * KCH, rem)]],
                    rows_r, sem))
                offs.append((rows_r, nfull * KCH, rem))
            for h, (buf, off, ln) in zip(hs, offs):
                h.wait()
                pltpu.sync_copy(buf, out_hbm.at[pl.ds(base + off, ln)])

    if with_dist:
        return k(table, pos_xyz[0], pos_xyz[1], pos_xyz[2], col, row)
    return k(table, col, row)


def _sc_scatter(m, col, zeros_tab):
    e = col.shape[0]
    per_w = e // NW
    nfull = per_w // KCH
    rem = per_w - nfull * KCH
    rows_per_tile = NPAD // 16
    mesh = plsc.VectorSubcoreMesh(core_axis_name="c", subcore_axis_name="s")

    scratch = [
        pltpu.VMEM((KCH,), jnp.int32),
        pltpu.VMEM((KCH, H), _F32),
        pltpu.VMEM_SHARED((NPAD, H), _F32),
        pltpu.SemaphoreType.DMA,
    ]
    if rem:
        scratch += [pltpu.VMEM((rem,), jnp.int32), pltpu.VMEM((rem, H), _F32)]

    @functools.partial(
        pl.kernel,
        mesh=mesh,
        out_type=jax.ShapeDtypeStruct((2, NPAD, H), _F32),
        scratch_types=scratch,
    )
    def k(m_hbm, col_hbm, zeros_hbm, parts_hbm, idx_v, rows_v, acc, sem,
          *rest):
        c = lax.axis_index("c")
        s = lax.axis_index("s")
        wid = s * 2 + c
        base = wid * per_w
        zoff = s * rows_per_tile
        pltpu.sync_copy(zeros_hbm.at[pl.ds(zoff, rows_per_tile)],
                        acc.at[pl.ds(zoff, rows_per_tile)])
        plsc.subcore_barrier()

        def body(ci, _):
            off = base + ci * KCH
            pltpu.sync_copy(col_hbm.at[pl.ds(off, KCH)], idx_v)
            pltpu.sync_copy(m_hbm.at[pl.ds(off, KCH)], rows_v)
            pltpu.sync_copy(rows_v, acc.at[idx_v], add=True)
            return 0
        lax.fori_loop(0, nfull, body, 0)
        if rem:
            idx_r, rows_r = rest
            off = base + nfull * KCH
            pltpu.sync_copy(col_hbm.at[pl.ds(off, rem)], idx_r)
            pltpu.sync_copy(m_hbm.at[pl.ds(off, rem)], rows_r)
            pltpu.sync_copy(rows_r, acc.at[idx_r], add=True)
        plsc.subcore_barrier()
        pltpu.sync_copy(acc.at[pl.ds(zoff, rows_per_tile)],
                        parts_hbm.at[c, pl.ds(zoff, rows_per_tile)])

    return k(m, col, zeros_tab)


def _full(shape):
    return pl.BlockSpec(shape, lambda i: (0,) * len(shape))


def _tc_prologue(at2, b2, t2, p):
    bn = 2048
    nat = p["atom_emb"].shape[0]
    nb = t2.shape[0]

    def body(at_ref, b_ref, t_ref, emb_w, t1w, t1b, t2w, t2b, h_ref):
        tcol = t_ref[...]
        io = lax.broadcasted_iota(jnp.int32, (1, H // 2), 1).astype(_F32)
        freqs = jnp.exp(io * (-math.log(10000.0) / (H // 2 - 1)))
        args = tcol * freqs
        emb = jnp.concatenate([jnp.sin(args), jnp.cos(args)], axis=1)
        temb = _dot(_silu(_dot(emb, t1w[...]) + t1b[...]), t2w[...]) + t2b[...]
        oh_a = (at_ref[...] == lax.broadcasted_iota(jnp.int32, (bn, nat), 1)
                ).astype(_F32)
        oh_b = (b_ref[...] == lax.broadcasted_iota(jnp.int32, (bn, nb), 1)
                ).astype(_F32)
        h_ref[...] = _dot(oh_a, emb_w[...]) + _dot(oh_b, temb)

    return pl.pallas_call(
        body,
        grid=(NPAD // bn,),
        in_specs=[
            pl.BlockSpec((bn, 1), lambda i: (i, 0)),
            pl.BlockSpec((bn, 1), lambda i: (i, 0)),
            _full((nb, 1)),
            _full(p["atom_emb"].shape),
            _full((H, H)), _full((1, H)),
            _full((H, H)), _full((1, H)),
        ],
        out_specs=pl.BlockSpec((bn, H), lambda i: (i, 0)),
        out_shape=jax.ShapeDtypeStruct((NPAD, H), _F32),
    )(at2, b2, t2, p["atom_emb"],
      p["time1"]["W"], p["time1"]["b"].reshape(1, H),
      p["time2"]["W"], p["time2"]["b"].reshape(1, H))


def _edge_tail(xi, xj, ea, lw):
    cat1 = jnp.concatenate([xi, xj, ea], axis=1)
    e1 = _silu(_dot(cat1, lw["e1w"][...]) + lw["e1b"][...])
    ef = _dot(e1, lw["e2w"][...]) + lw["e2b"][...]
    cat2 = jnp.concatenate([xi, xj, ef], axis=1)
    n1 = _silu(_dot(cat2, lw["n1w"][...]) + lw["n1b"][...])
    return _dot(n1, lw["n2w"][...]) + lw["n2b"][...]


def _layer_weights(lp):
    return {
        "e1w": lp["edge1"]["W"], "e1b": lp["edge1"]["b"].reshape(1, EC),
        "e2w": lp["edge2"]["W"], "e2b": lp["edge2"]["b"].reshape(1, EC),
        "n1w": lp["node1"]["W"], "n1b": lp["node1"]["b"].reshape(1, 2 * H),
        "n2w": lp["node2"]["W"], "n2b": lp["node2"]["b"].reshape(1, H),
    }


_LW_ORDER = ("e1w", "e1b", "e2w", "e2b", "n1w", "n1b", "n2w", "n2b")


def _edge_block(e):
    return 2048 if e % 2048 == 0 else e // 32


def _tc_edge1(Xi, Xj, d2, lp, p):
    e = Xi.shape[0]
    be = _edge_block(e)
    lw = _layer_weights(lp)

    def body(xi_ref, xj_ref, d2_ref, *refs):
        lwr = dict(zip(_LW_ORDER, refs[:8]))
        ec1w, ec1b, ec2w, ec2b, m_ref, ea_ref = refs[8:]
        dist = jnp.sqrt(d2_ref[...])
        ea0 = _silu(dist * ec1w[...] + ec1b[...])
        ea = _dot(ea0, ec2w[...]) + ec2b[...]
        ea_ref[...] = ea
        m_ref[...] = _edge_tail(xi_ref[...], xj_ref[...], ea, lwr)

    lw_vals = [lw[k] for k in _LW_ORDER]
    return pl.pallas_call(
        body,
        grid=(e // be,),
        in_specs=[pl.BlockSpec((be, H), lambda i: (i, 0))] * 2
        + [pl.BlockSpec((be, 1), lambda i: (i, 0))]
        + [_full(v.shape) for v in lw_vals]
        + [_full((1, EC))] * 2 + [_full((EC, EC)), _full((1, EC))],
        out_specs=[pl.BlockSpec((be, H), lambda i: (i, 0)),
                   pl.BlockSpec((be, EC), lambda i: (i, 0))],
        out_shape=[jax.ShapeDtypeStruct((e, H), _F32),
                   jax.ShapeDtypeStruct((e, EC), _F32)],
    )(Xi, Xj, d2, *lw_vals,
      p["edge_enc1"]["W"].reshape(1, EC), p["edge_enc1"]["b"].reshape(1, EC),
      p["edge_enc2"]["W"], p["edge_enc2"]["b"].reshape(1, EC))


def _tc_edge2(Xi, Xj, ea_arr, lp):
    e = Xi.shape[0]
    be = _edge_block(e)
    lw = _layer_weights(lp)

    def body(xi_ref, xj_ref, ea_ref, *refs):
        lwr = dict(zip(_LW_ORDER, refs[:8]))
        m_ref = refs[8]
        m_ref[...] = _edge_tail(xi_ref[...], xj_ref[...], ea_ref[...], lwr)

    lw_vals = [lw[k] for k in _LW_ORDER]
    return pl.pallas_call(
        body,
        grid=(e // be,),
        in_specs=[pl.BlockSpec((be, H), lambda i: (i, 0))] * 2
        + [pl.BlockSpec((be, EC), lambda i: (i, 0))]
        + [_full(v.shape) for v in lw_vals],
        out_specs=pl.BlockSpec((be, H), lambda i: (i, 0)),
        out_shape=jax.ShapeDtypeStruct((e, H), _F32),
    )(Xi, Xj, ea_arr, *lw_vals)


def _tc_update(table, parts_a, parts_b):
    bn = 2048

    def body(t_ref, a0, a1, b0, b1, o_ref):
        o_ref[...] = (t_ref[...] + a0[...][0] + a1[...][0]
                      + b0[...][0] + b1[...][0])

    return pl.pallas_call(
        body,
        grid=(NPAD // bn,),
        in_specs=[pl.BlockSpec((bn, H), lambda i: (i, 0)),
                  pl.BlockSpec((1, bn, H), lambda i: (0, i, 0)),
                  pl.BlockSpec((1, bn, H), lambda i: (1, i, 0)),
                  pl.BlockSpec((1, bn, H), lambda i: (0, i, 0)),
                  pl.BlockSpec((1, bn, H), lambda i: (1, i, 0))],
        out_specs=pl.BlockSpec((bn, H), lambda i: (i, 0)),
        out_shape=jax.ShapeDtypeStruct((NPAD, H), _F32),
    )(table, parts_a, parts_a, parts_b, parts_b)


def _tc_epilogue(table, parts_a, parts_b, p, n):
    bn = 2000
    nat_out = p["atom2"]["W"].shape[1]

    def body(t_ref, a0, a1, b0, b1, wp1, bp1, wp2, bp2, wa1, ba1, wa2, ba2,
             pn_ref, al_ref):
        h = (t_ref[...] + a0[...][0] + a1[...][0]
             + b0[...][0] + b1[...][0])
        pn_ref[...] = _dot(_silu(_dot(h, wp1[...]) + bp1[...]),
                           wp2[...]) + bp2[...]
        al_ref[...] = _dot(_silu(_dot(h, wa1[...]) + ba1[...]),
                           wa2[...]) + ba2[...]

    return pl.pallas_call(
        body,
        grid=(n // bn,),
        in_specs=[pl.BlockSpec((bn, H), lambda i: (i, 0)),
                  pl.BlockSpec((1, bn, H), lambda i: (0, i, 0)),
                  pl.BlockSpec((1, bn, H), lambda i: (1, i, 0)),
                  pl.BlockSpec((1, bn, H), lambda i: (0, i, 0)),
                  pl.BlockSpec((1, bn, H), lambda i: (1, i, 0))]
        + [_full((H, H)), _full((1, H)), _full((H, 3)), _full((1, 3)),
           _full((H, H)), _full((1, H)), _full((H, nat_out)),
           _full((1, nat_out))],
        out_specs=[pl.BlockSpec((bn, 3), lambda i: (i, 0)),
                   pl.BlockSpec((bn, nat_out), lambda i: (i, 0))],
        out_shape=[jax.ShapeDtypeStruct((n, 3), _F32),
                   jax.ShapeDtypeStruct((n, nat_out), _F32)],
    )(table, parts_a, parts_a, parts_b, parts_b,
      p["pos1"]["W"], p["pos1"]["b"].reshape(1, H),
      p["pos2"]["W"], p["pos2"]["b"].reshape(1, 3),
      p["atom1"]["W"], p["atom1"]["b"].reshape(1, H),
      p["atom2"]["W"], p["atom2"]["b"].reshape(1, nat_out))


def kernel(x, pos, params, atom_type, batch, t, edge_index):
    del x
    p = params
    n = pos.shape[0]
    pad = NPAD - n
    row = edge_index[0].astype(jnp.int32)
    col = edge_index[1].astype(jnp.int32)
    at2 = jnp.pad(atom_type.astype(jnp.int32), (0, pad)).reshape(NPAD, 1)
    b2 = jnp.pad(batch.astype(jnp.int32), (0, pad)).reshape(NPAD, 1)
    pos3 = jnp.pad(pos.astype(_F32), ((0, pad), (0, 0)))
    pos_xyz = (pos3[:, 0], pos3[:, 1], pos3[:, 2])
    t2 = t.astype(_F32).reshape(-1, 1)
    zeros_tab = jnp.zeros((NPAD, H), _F32)

    table = _tc_prologue(at2, b2, t2, p)

    e = col.shape[0]
    ea_split = (e // 2) // 256 * 256
    halves = [(col[:ea_split], row[:ea_split]),
              (col[ea_split:], row[ea_split:])]

    ea = [None, None]
    num_layers = len(p["layers"])
    for li, lp in enumerate(p["layers"]):
        parts = []
        for hi, (colh, rowh) in enumerate(halves):
            if li == 0:
                Xi, Xj, d2 = _sc_gather(table, colh, rowh, pos_xyz=pos_xyz)
                M, ea[hi] = _tc_edge1(Xi, Xj, d2.reshape(-1, 1), lp, p)
            else:
                Xi, Xj = _sc_gather(table, colh, rowh)
                M = _tc_edge2(Xi, Xj, ea[hi], lp)
            parts.append(_sc_scatter(M, colh, zeros_tab))
        if li < num_layers - 1:
            table = _tc_update(table, parts[0], parts[1])
        else:
            out = _tc_epilogue(table, parts[0], parts[1], p, n)
    return out


# speedup vs baseline: 4.0363x; 1.0273x over previous
import functools
import math

import jax
import jax.numpy as jnp
from jax import lax
from jax.experimental import pallas as pl
from jax.experimental.pallas import tpu as pltpu
from jax.experimental.pallas import tpu_sc as plsc

H = 128
EC = 16
PW = 128
NW = 32
KCH = 128
NPAD = 10240

_F32 = jnp.float32


def _silu(v):
    return v * (1.0 / (1.0 + jnp.exp(-v)))


def _dot(a, b):
    return jnp.dot(a, b, preferred_element_type=_F32)


def _sc_gather(table, col, row, pos_xyz=None):
    e = col.shape[0]
    per_w = e // NW
    nfull = per_w // KCH
    rem = per_w - nfull * KCH
    nb = 4
    ngrp = nfull // nb
    ltail = nfull - ngrp * nb
    with_dist = pos_xyz is not None
    npad = table.shape[0]
    mesh = plsc.VectorSubcoreMesh(core_axis_name="c", subcore_axis_name="s")

    scratch = [
        pltpu.VMEM((per_w,), jnp.int32),
        pltpu.VMEM((per_w,), jnp.int32),
    ] + [pltpu.VMEM((KCH, H), _F32) for _ in range(nb)] + [
        pltpu.SemaphoreType.DMA,
    ]
    out_type = [
        jax.ShapeDtypeStruct((e, H), _F32),
        jax.ShapeDtypeStruct((e, H), _F32),
    ]
    if rem:
        scratch.append(pltpu.VMEM((rem, H), _F32))
    if with_dist:
        scratch += [
            pltpu.VMEM((npad,), _F32),
            pltpu.VMEM((npad,), _F32),
            pltpu.VMEM((npad,), _F32),
            pltpu.VMEM((per_w,), _F32),
        ]
        out_type.append(jax.ShapeDtypeStruct((e,), _F32))

    @functools.partial(
        pl.kernel,
        mesh=mesh,
        out_type=tuple(out_type),
        scratch_types=scratch,
        compiler_params=pltpu.CompilerParams(needs_layout_passes=False),
    )
    def k(*refs):
        if with_dist:
            (table_hbm, px_hbm, py_hbm, pz_hbm, col_hbm, row_hbm,
             xi_hbm, xj_hbm, d2_hbm) = refs[:9]
            rest = refs[9:]
        else:
            table_hbm, col_hbm, row_hbm, xi_hbm, xj_hbm = refs[:5]
            rest = refs[5:]
        ic_v, ir_v = rest[:2]
        rows = rest[2:2 + nb]
        sem = rest[2 + nb]
        rest = rest[3 + nb:]
        if rem:
            rows_r = rest[0]
            rest = rest[1:]
        if with_dist:
            px_t, py_t, pz_t, d2_v = rest

        wid = lax.axis_index("s") * 2 + lax.axis_index("c")
        base = wid * per_w
        pltpu.sync_copy(col_hbm.at[pl.ds(base, per_w)], ic_v)
        pltpu.sync_copy(row_hbm.at[pl.ds(base, per_w)], ir_v)

        if with_dist:
            pltpu.sync_copy(px_hbm, px_t)
            pltpu.sync_copy(py_hbm, py_t)
            pltpu.sync_copy(pz_hbm, pz_t)

            def d2_at(s):
                ic = ic_v[pl.ds(s, 16)]
                ir = ir_v[pl.ds(s, 16)]
                dx = plsc.load_gather(px_t, [ir]) - plsc.load_gather(px_t, [ic])
                dy = plsc.load_gather(py_t, [ir]) - plsc.load_gather(py_t, [ic])
                dz = plsc.load_gather(pz_t, [ir]) - plsc.load_gather(pz_t, [ic])
                d2_v[pl.ds(s, 16)] = dx * dx + dy * dy + dz * dz

            def dbody(kk, _):
                d2_at(kk * 16)
                return 0
            lax.fori_loop(0, per_w // 16, dbody, 0)
            if per_w % 16:
                d2_at(per_w - 16)
            pltpu.sync_copy(d2_v, d2_hbm.at[pl.ds(base, per_w)])

        for idx_big, out_hbm in ((ic_v, xi_hbm), (ir_v, xj_hbm)):
            def grp(g, _, idx_big=idx_big, out_hbm=out_hbm):
                hs = [
                    pltpu.async_copy(
                        table_hbm.at[idx_big.at[pl.ds((g * nb + j) * KCH, KCH)]],
                        rows[j], sem)
                    for j in range(nb)
                ]
                for j, h in enumerate(hs):
                    h.wait()
                    pltpu.sync_copy(
                        rows[j],
                        out_hbm.at[pl.ds(base + (g * nb + j) * KCH, KCH)])
                return 0
            lax.fori_loop(0, ngrp, grp, 0)
            hs = []
            offs = []
            for j in range(ltail):
                ci = ngrp * nb + j
                hs.append(pltpu.async_copy(
                    table_hbm.at[idx_big.at[pl.ds(ci * KCH, KCH)]],
                    rows[j], sem))
                offs.append((rows[j], ci * KCH, KCH))
            if rem:
                hs.append(pltpu.async_copy(
                    table_hbm.at[idx_big.at[pl.ds(nfull * KCH, rem)]],
                    rows_r, sem))
                offs.append((rows_r, nfull * KCH, rem))
            for h, (buf, off, ln) in zip(hs, offs):
                h.wait()
                pltpu.sync_copy(buf, out_hbm.at[pl.ds(base + off, ln)])

    if with_dist:
        return k(table, pos_xyz[0], pos_xyz[1], pos_xyz[2], col, row)
    return k(table, col, row)


def _sc_scatter(m, col, zeros_tab):
    e = col.shape[0]
    per_w = e // NW
    nfull = per_w // KCH
    rem = per_w - nfull * KCH
    rows_per_tile = NPAD // 16
    mesh = plsc.VectorSubcoreMesh(core_axis_name="c", subcore_axis_name="s")

    nb = 2
    ngrp = nfull // nb
    ltail = nfull - ngrp * nb
    scratch = (
        [pltpu.VMEM((KCH,), jnp.int32) for _ in range(nb)]
        + [pltpu.VMEM((KCH, H), _F32) for _ in range(nb)]
        + [pltpu.VMEM_SHARED((NPAD, H), _F32), pltpu.SemaphoreType.DMA]
    )
    if rem:
        scratch += [pltpu.VMEM((rem,), jnp.int32), pltpu.VMEM((rem, H), _F32)]

    @functools.partial(
        pl.kernel,
        mesh=mesh,
        out_type=jax.ShapeDtypeStruct((2, NPAD, H), _F32),
        scratch_types=scratch,
    )
    def k(m_hbm, col_hbm, zeros_hbm, parts_hbm, *rest):
        idxb = rest[:nb]
        mb = rest[nb:2 * nb]
        acc, sem = rest[2 * nb:2 * nb + 2]
        rest = rest[2 * nb + 2:]
        c = lax.axis_index("c")
        s = lax.axis_index("s")
        wid = s * 2 + c
        base = wid * per_w
        zoff = s * rows_per_tile
        pltpu.sync_copy(zeros_hbm.at[pl.ds(zoff, rows_per_tile)],
                        acc.at[pl.ds(zoff, rows_per_tile)])
        plsc.subcore_barrier()

        def grp(g, _):
            hs = []
            for j in range(nb):
                off = base + (g * nb + j) * KCH
                pltpu.sync_copy(col_hbm.at[pl.ds(off, KCH)], idxb[j])
                hs.append(pltpu.async_copy(m_hbm.at[pl.ds(off, KCH)], mb[j],
                                           sem))
            for j, h in enumerate(hs):
                h.wait()
                pltpu.sync_copy(mb[j], acc.at[idxb[j]], add=True)
            return 0
        lax.fori_loop(0, ngrp, grp, 0)
        hs = []
        tails = []
        for j in range(ltail):
            off = base + (ngrp * nb + j) * KCH
            pltpu.sync_copy(col_hbm.at[pl.ds(off, KCH)], idxb[j])
            hs.append(pltpu.async_copy(m_hbm.at[pl.ds(off, KCH)], mb[j], sem))
            tails.append((mb[j], idxb[j]))
        if rem:
            idx_r, rows_r = rest
            off = base + nfull * KCH
            pltpu.sync_copy(col_hbm.at[pl.ds(off, rem)], idx_r)
            hs.append(pltpu.async_copy(m_hbm.at[pl.ds(off, rem)], rows_r, sem))
            tails.append((rows_r, idx_r))
        for h, (buf, ib) in zip(hs, tails):
            h.wait()
            pltpu.sync_copy(buf, acc.at[ib], add=True)
        plsc.subcore_barrier()
        pltpu.sync_copy(acc.at[pl.ds(zoff, rows_per_tile)],
                        parts_hbm.at[c, pl.ds(zoff, rows_per_tile)])

    return k(m, col, zeros_tab)


def _full(shape):
    return pl.BlockSpec(shape, lambda i: (0,) * len(shape))


def _tc_prologue(at2, b2, t2, p):
    bn = 2048
    nat = p["atom_emb"].shape[0]
    nb = t2.shape[0]

    def body(at_ref, b_ref, t_ref, emb_w, t1w, t1b, t2w, t2b, h_ref):
        tcol = t_ref[...]
        io = lax.broadcasted_iota(jnp.int32, (1, H // 2), 1).astype(_F32)
        freqs = jnp.exp(io * (-math.log(10000.0) / (H // 2 - 1)))
        args = tcol * freqs
        emb = jnp.concatenate([jnp.sin(args), jnp.cos(args)], axis=1)
        temb = _dot(_silu(_dot(emb, t1w[...]) + t1b[...]), t2w[...]) + t2b[...]
        oh_a = (at_ref[...] == lax.broadcasted_iota(jnp.int32, (bn, nat), 1)
                ).astype(_F32)
        oh_b = (b_ref[...] == lax.broadcasted_iota(jnp.int32, (bn, nb), 1)
                ).astype(_F32)
        h_ref[...] = _dot(oh_a, emb_w[...]) + _dot(oh_b, temb)

    return pl.pallas_call(
        body,
        grid=(NPAD // bn,),
        in_specs=[
            pl.BlockSpec((bn, 1), lambda i: (i, 0)),
            pl.BlockSpec((bn, 1), lambda i: (i, 0)),
            _full((nb, 1)),
            _full(p["atom_emb"].shape),
            _full((H, H)), _full((1, H)),
            _full((H, H)), _full((1, H)),
        ],
        out_specs=pl.BlockSpec((bn, H), lambda i: (i, 0)),
        out_shape=jax.ShapeDtypeStruct((NPAD, H), _F32),
    )(at2, b2, t2, p["atom_emb"],
      p["time1"]["W"], p["time1"]["b"].reshape(1, H),
      p["time2"]["W"], p["time2"]["b"].reshape(1, H))


def _edge_tail(xi, xj, ea, lw):
    cat1 = jnp.concatenate([xi, xj, ea], axis=1)
    e1 = _silu(_dot(cat1, lw["e1w"][...]) + lw["e1b"][...])
    ef = _dot(e1, lw["e2w"][...]) + lw["e2b"][...]
    cat2 = jnp.concatenate([xi, xj, ef], axis=1)
    n1 = _silu(_dot(cat2, lw["n1w"][...]) + lw["n1b"][...])
    return _dot(n1, lw["n2w"][...]) + lw["n2b"][...]


def _layer_weights(lp):
    return {
        "e1w": lp["edge1"]["W"], "e1b": lp["edge1"]["b"].reshape(1, EC),
        "e2w": lp["edge2"]["W"], "e2b": lp["edge2"]["b"].reshape(1, EC),
        "n1w": lp["node1"]["W"], "n1b": lp["node1"]["b"].reshape(1, 2 * H),
        "n2w": lp["node2"]["W"], "n2b": lp["node2"]["b"].reshape(1, H),
    }


_LW_ORDER = ("e1w", "e1b", "e2w", "e2b", "n1w", "n1b", "n2w", "n2b")


def _edge_block(e):
    return 2048 if e % 2048 == 0 else e // 32


def _tc_edge1(Xi, Xj, d2, lp, p):
    e = Xi.shape[0]
    be = _edge_block(e)
    lw = _layer_weights(lp)

    def body(xi_ref, xj_ref, d2_ref, *refs):
        lwr = dict(zip(_LW_ORDER, refs[:8]))
        ec1w, ec1b, ec2w, ec2b, m_ref, ea_ref = refs[8:]
        dist = jnp.sqrt(d2_ref[...])
        ea0 = _silu(dist * ec1w[...] + ec1b[...])
        ea = _dot(ea0, ec2w[...]) + ec2b[...]
        ea_ref[...] = ea
        m_ref[...] = _edge_tail(xi_ref[...], xj_ref[...], ea, lwr)

    lw_vals = [lw[k] for k in _LW_ORDER]
    return pl.pallas_call(
        body,
        grid=(e // be,),
        in_specs=[pl.BlockSpec((be, H), lambda i: (i, 0))] * 2
        + [pl.BlockSpec((be, 1), lambda i: (i, 0))]
        + [_full(v.shape) for v in lw_vals]
        + [_full((1, EC))] * 2 + [_full((EC, EC)), _full((1, EC))],
        out_specs=[pl.BlockSpec((be, H), lambda i: (i, 0)),
                   pl.BlockSpec((be, EC), lambda i: (i, 0))],
        out_shape=[jax.ShapeDtypeStruct((e, H), _F32),
                   jax.ShapeDtypeStruct((e, EC), _F32)],
    )(Xi, Xj, d2, *lw_vals,
      p["edge_enc1"]["W"].reshape(1, EC), p["edge_enc1"]["b"].reshape(1, EC),
      p["edge_enc2"]["W"], p["edge_enc2"]["b"].reshape(1, EC))


def _tc_edge2(Xi, Xj, ea_arr, lp):
    e = Xi.shape[0]
    be = _edge_block(e)
    lw = _layer_weights(lp)

    def body(xi_ref, xj_ref, ea_ref, *refs):
        lwr = dict(zip(_LW_ORDER, refs[:8]))
        m_ref = refs[8]
        m_ref[...] = _edge_tail(xi_ref[...], xj_ref[...], ea_ref[...], lwr)

    lw_vals = [lw[k] for k in _LW_ORDER]
    return pl.pallas_call(
        body,
        grid=(e // be,),
        in_specs=[pl.BlockSpec((be, H), lambda i: (i, 0))] * 2
        + [pl.BlockSpec((be, EC), lambda i: (i, 0))]
        + [_full(v.shape) for v in lw_vals],
        out_specs=pl.BlockSpec((be, H), lambda i: (i, 0)),
        out_shape=jax.ShapeDtypeStruct((e, H), _F32),
    )(Xi, Xj, ea_arr, *lw_vals)


def _tc_update(table, parts_a, parts_b):
    bn = 2048

    def body(t_ref, a0, a1, b0, b1, o_ref):
        o_ref[...] = (t_ref[...] + a0[...][0] + a1[...][0]
                      + b0[...][0] + b1[...][0])

    return pl.pallas_call(
        body,
        grid=(NPAD // bn,),
        in_specs=[pl.BlockSpec((bn, H), lambda i: (i, 0)),
                  pl.BlockSpec((1, bn, H), lambda i: (0, i, 0)),
                  pl.BlockSpec((1, bn, H), lambda i: (1, i, 0)),
                  pl.BlockSpec((1, bn, H), lambda i: (0, i, 0)),
                  pl.BlockSpec((1, bn, H), lambda i: (1, i, 0))],
        out_specs=pl.BlockSpec((bn, H), lambda i: (i, 0)),
        out_shape=jax.ShapeDtypeStruct((NPAD, H), _F32),
    )(table, parts_a, parts_a, parts_b, parts_b)


def _tc_epilogue(table, parts_a, parts_b, p, n):
    bn = 2000
    nat_out = p["atom2"]["W"].shape[1]

    def body(t_ref, a0, a1, b0, b1, wp1, bp1, wp2, bp2, wa1, ba1, wa2, ba2,
             pn_ref, al_ref):
        h = (t_ref[...] + a0[...][0] + a1[...][0]
             + b0[...][0] + b1[...][0])
        pn_ref[...] = _dot(_silu(_dot(h, wp1[...]) + bp1[...]),
                           wp2[...]) + bp2[...]
        al_ref[...] = _dot(_silu(_dot(h, wa1[...]) + ba1[...]),
                           wa2[...]) + ba2[...]

    return pl.pallas_call(
        body,
        grid=(n // bn,),
        in_specs=[pl.BlockSpec((bn, H), lambda i: (i, 0)),
                  pl.BlockSpec((1, bn, H), lambda i: (0, i, 0)),
                  pl.BlockSpec((1, bn, H), lambda i: (1, i, 0)),
                  pl.BlockSpec((1, bn, H), lambda i: (0, i, 0)),
                  pl.BlockSpec((1, bn, H), lambda i: (1, i, 0))]
        + [_full((H, H)), _full((1, H)), _full((H, 3)), _full((1, 3)),
           _full((H, H)), _full((1, H)), _full((H, nat_out)),
           _full((1, nat_out))],
        out_specs=[pl.BlockSpec((bn, 3), lambda i: (i, 0)),
                   pl.BlockSpec((bn, nat_out), lambda i: (i, 0))],
        out_shape=[jax.ShapeDtypeStruct((n, 3), _F32),
                   jax.ShapeDtypeStruct((n, nat_out), _F32)],
    )(table, parts_a, parts_a, parts_b, parts_b,
      p["pos1"]["W"], p["pos1"]["b"].reshape(1, H),
      p["pos2"]["W"], p["pos2"]["b"].reshape(1, 3),
      p["atom1"]["W"], p["atom1"]["b"].reshape(1, H),
      p["atom2"]["W"], p["atom2"]["b"].reshape(1, nat_out))


def kernel(x, pos, params, atom_type, batch, t, edge_index):
    del x
    p = params
    n = pos.shape[0]
    pad = NPAD - n
    row = edge_index[0].astype(jnp.int32)
    col = edge_index[1].astype(jnp.int32)
    at2 = jnp.pad(atom_type.astype(jnp.int32), (0, pad)).reshape(NPAD, 1)
    b2 = jnp.pad(batch.astype(jnp.int32), (0, pad)).reshape(NPAD, 1)
    pos3 = jnp.pad(pos.astype(_F32), ((0, pad), (0, 0)))
    pos_xyz = (pos3[:, 0], pos3[:, 1], pos3[:, 2])
    t2 = t.astype(_F32).reshape(-1, 1)
    zeros_tab = jnp.zeros((NPAD, H), _F32)

    table = _tc_prologue(at2, b2, t2, p)

    e = col.shape[0]
    ea_split = (e // 2) // 256 * 256
    halves = [(col[:ea_split], row[:ea_split]),
              (col[ea_split:], row[ea_split:])]

    ea = [None, None]
    num_layers = len(p["layers"])
    for li, lp in enumerate(p["layers"]):
        parts = []
        for hi, (colh, rowh) in enumerate(halves):
            if li == 0:
                Xi, Xj, d2 = _sc_gather(table, colh, rowh, pos_xyz=pos_xyz)
                M, ea[hi] = _tc_edge1(Xi, Xj, d2.reshape(-1, 1), lp, p)
            else:
                Xi, Xj = _sc_gather(table, colh, rowh)
                M = _tc_edge2(Xi, Xj, ea[hi], lp)
            parts.append(_sc_scatter(M, colh, zeros_tab))
        if li < num_layers - 1:
            table = _tc_update(table, parts[0], parts[1])
        else:
            out = _tc_epilogue(table, parts[0], parts[1], p, n)
    return out
